# Initial kernel scaffold; baseline (speedup 1.0000x reference)
import jax
import jax.numpy as jnp
from jax.experimental import pallas as pl


def kernel(h, coord, edge_attr, edge_index, We1, be1, We2, be2, Wn1, bn1, Wn2, bn2, Wc1, bc1, Wc2, A, w):
    raise NotImplementedError("write your pallas kernel here")


# speedup vs baseline: 38.7932x; 38.7932x over previous
import functools
import jax
import jax.numpy as jnp
from jax import lax
from jax.experimental import pallas as pl
from jax.experimental.pallas import tpu as pltpu
from jax.experimental.pallas import tpu_sc as plsc

_N = 10000
_DEG = 16
_F = 128
_H = 128
_DE = 16
_E = _N * _DEG

_BN = 40
_BE = _BN * _DEG
_GRID = _N // _BN

_PAIRS = (
    (0, 1), (2, 3), (4, 5), (6, 7), (8, 9), (10, 11), (12, 13), (14, 15),
    (0, 2), (1, 3), (4, 6), (5, 7), (8, 10), (9, 11), (12, 14), (13, 15),
    (1, 2), (5, 6), (9, 10), (13, 14), (0, 4), (1, 5), (2, 6), (3, 7),
    (8, 12), (9, 13), (10, 14), (11, 15), (2, 4), (3, 5), (10, 12), (11, 13),
    (1, 2), (3, 4), (5, 6), (9, 10), (11, 12), (13, 14), (0, 8), (1, 9),
    (2, 10), (3, 11), (4, 12), (5, 13), (6, 14), (7, 15), (4, 8), (5, 9),
    (6, 10), (7, 11), (2, 4), (3, 5), (6, 8), (7, 9), (10, 12), (11, 13),
    (1, 2), (3, 4), (5, 6), (7, 8), (9, 10), (11, 12), (13, 14),
)


def _silu(x):
    return x * (1.0 / (1.0 + jnp.exp(-x)))


def _ssqrt(sq):
    pos = sq > 1e-12
    return jnp.where(pos, jnp.sqrt(jnp.where(pos, sq, 1.0)), 0.0)


def _dot(a, b):
    return lax.dot_general(a, b, (((1,), (0,)), ((), ())),
                           preferred_element_type=jnp.float32)


_NC = 2
_NS = 16
_NW = _NC * _NS
_CH = 128
_NCHUNK = _E // _CH
_MAXIT = (_NCHUNK + _NW - 1) // _NW

_sc_gather_built = None


def _get_sc_gather():
    global _sc_gather_built
    if _sc_gather_built is not None:
        return _sc_gather_built
    mesh = plsc.VectorSubcoreMesh(core_axis_name="c", subcore_axis_name="s")

    @functools.partial(
        pl.kernel,
        out_type=(jax.ShapeDtypeStruct((_E, _F), jnp.float32),
                  jax.ShapeDtypeStruct((_E,), jnp.float32),
                  jax.ShapeDtypeStruct((_E,), jnp.float32),
                  jax.ShapeDtypeStruct((_E,), jnp.float32)),
        mesh=mesh,
        scratch_types=[
            pltpu.VMEM((_CH,), jnp.int32),
            pltpu.VMEM((_CH, _F), jnp.float32),
            pltpu.VMEM((_CH,), jnp.float32),
            pltpu.VMEM((_CH,), jnp.float32),
            pltpu.VMEM((_CH,), jnp.float32),
            pltpu.VMEM((_N,), jnp.float32),
            pltpu.VMEM((_N,), jnp.float32),
            pltpu.VMEM((_N,), jnp.float32),
            pltpu.SemaphoreType.DMA,
        ],
        compiler_params=pltpu.CompilerParams(needs_layout_passes=False),
    )
    def _sc_gather(h_hbm, cx_hbm, cy_hbm, cz_hbm, col_hbm,
                   hc_out, ccx_out, ccy_out, ccz_out,
                   idx_v, hrows_v, gx_v, gy_v, gz_v, cx_v, cy_v, cz_v, s1):
        wid = lax.axis_index("s") * _NC + lax.axis_index("c")
        pltpu.sync_copy(cx_hbm, cx_v)
        pltpu.sync_copy(cy_hbm, cy_v)
        pltpu.sync_copy(cz_hbm, cz_v)

        def body(t, carry):
            cid = wid + t * _NW

            @pl.when(cid < _NCHUNK)
            def _():
                off = cid * _CH
                pltpu.sync_copy(col_hbm.at[pl.ds(off, _CH)], idx_v)
                cp1 = pltpu.async_copy(h_hbm.at[idx_v], hrows_v, s1)
                for j in range(_CH // 16):
                    sl = pl.ds(j * 16, 16)
                    ii = idx_v[sl]
                    gx_v[sl] = plsc.load_gather(cx_v, [ii])
                    gy_v[sl] = plsc.load_gather(cy_v, [ii])
                    gz_v[sl] = plsc.load_gather(cz_v, [ii])
                cp1.wait()
                pltpu.sync_copy(hrows_v, hc_out.at[pl.ds(off, _CH)])
                pltpu.sync_copy(gx_v, ccx_out.at[pl.ds(off, _CH)])
                pltpu.sync_copy(gy_v, ccy_out.at[pl.ds(off, _CH)])
                pltpu.sync_copy(gz_v, ccz_out.at[pl.ds(off, _CH)])

            return carry

        lax.fori_loop(0, _MAXIT, body, 0, unroll=False)

    _sc_gather_built = _sc_gather
    return _sc_gather_built


def _tc_body(h_ref, coord_ref, cjx_ref, cjy_ref, cjz_ref,
             ncx_ref, ncy_ref, ncz_ref, hc_ref,
             ea_ref, We1a, We1b, Wl6, We1g, We1e, b1, We2, b2, Wc1, bc1r,
             wc2r, Apad, wT, Wn1a, Wn1b, bn1r, Wn2, bn2r,
             hout_ref, cout_ref):
    def bcast(a):
        bn, k = a.shape
        return jnp.broadcast_to(a[:, None, :], (bn, _DEG, k)).reshape(bn * _DEG, k)

    def d3(ax, ay, az, bx, by, bz):
        return _ssqrt((ax - bx) ** 2 + (ay - by) ** 2 + (az - bz) ** 2)

    hi = h_ref[...]
    cie = bcast(coord_ref[...])
    cix, ciy, ciz = cie[:, 0:1], cie[:, 1:2], cie[:, 2:3]
    cjx, cjy, cjz = cjx_ref[...], cjy_ref[...], cjz_ref[...]
    px = ciy * cjz - ciz * cjy
    py = ciz * cjx - cix * cjz
    pz = cix * cjy - ciy * cjx
    Nx = bcast(ncx_ref[...])
    Ny = bcast(ncy_ref[...])
    Nz = bcast(ncz_ref[...])
    X0 = d3(px, py, pz, Nx, Ny, Nz)
    X1 = d3(cix, ciy, ciz, Nx, Ny, Nz)
    X2 = d3(cjx, cjy, cjz, Nx, Ny, Nz)
    n_p = _ssqrt(px * px + py * py + pz * pz)
    n_i = _ssqrt(cix * cix + ciy * ciy + ciz * ciz)
    n_j = _ssqrt(cjx * cjx + cjy * cjy + cjz * cjz)
    d_pi = d3(px, py, pz, cix, ciy, ciz)
    d_pj = d3(px, py, pz, cjx, cjy, cjz)
    d_ij = d3(cix, ciy, ciz, cjx, cjy, cjz)
    loc = (n_p * Wl6[0:1, :] + d_pi * Wl6[1:2, :] + d_pj * Wl6[2:3, :]
           + n_i * Wl6[3:4, :] + d_ij * Wl6[4:5, :] + n_j * Wl6[5:6, :])
    A0, A1, A2 = Apad[0:1, :], Apad[1:2, :], Apad[2:3, :]
    P = [X0[:, n:n + 1] * A0 + X1[:, n:n + 1] * A1 + X2[:, n:n + 1] * A2
         for n in range(_DEG)]
    for a, b in _PAIRS:
        lo = jnp.minimum(P[a], P[b])
        hi2 = jnp.maximum(P[a], P[b])
        P[a] = lo
        P[b] = hi2
    geo = P[0] * wT[0:1, :]
    for n in range(1, _DEG):
        geo = geo + P[n] * wT[n:n + 1, :]
    t1 = bcast(_dot(hi, We1a[...]))
    hid = _silu(t1 + _dot(hc_ref[...], We1b[...]) + loc
                + _dot(geo, We1g[...]) + _dot(ea_ref[...], We1e[...])
                + b1[...])
    ef = _silu(_dot(hid, We2[...]) + b2[...])
    u = _silu(_dot(ef, Wc1[...]) + bc1r[...])
    cs = jnp.sum(u * wc2r[...], axis=1, keepdims=True)
    tx = (cix - cjx) * cs
    ty = (ciy - cjy) * cs
    tz = (ciz - cjz) * cs
    inv = 1.0 / _DEG
    aggx = jnp.sum(tx.reshape(_BN, _DEG, 1), axis=1) * inv
    aggy = jnp.sum(ty.reshape(_BN, _DEG, 1), axis=1) * inv
    aggz = jnp.sum(tz.reshape(_BN, _DEG, 1), axis=1) * inv
    cout_ref[...] = coord_ref[...] + jnp.concatenate([aggx, aggy, aggz], axis=1)
    aggh = jnp.sum(ef.reshape(_BN, _DEG, _H), axis=1)
    nh = _silu(_dot(hi, Wn1a[...]) + _dot(aggh, Wn1b[...]) + bn1r[...])
    hout_ref[...] = hi + _dot(nh, Wn2[...]) + bn2r[...]


def _tc_specs():
    def nodes(k):
        return pl.BlockSpec((_BN, k), lambda i: (i, 0))

    def edges(k):
        return pl.BlockSpec((_BE, k), lambda i: (i, 0))

    def full(s):
        return pl.BlockSpec(s, lambda i: (0, 0))

    in_specs = [
        nodes(_F),
        nodes(3),
        edges(1),
        edges(1),
        edges(1),
        nodes(_DEG),
        nodes(_DEG),
        nodes(_DEG),
        edges(_F),
        edges(_DE),
        full((_F, _H)),
        full((_F, _H)),
        full((8, _H)),
        full((_H, _H)),
        full((_DE, _H)),
        full((1, _H)),
        full((_H, _H)),
        full((1, _H)),
        full((_H, _H)),
        full((1, _H)),
        full((1, _H)),
        full((8, _H)),
        full((_DEG, _H)),
        full((_F, _H)),
        full((_F, _H)),
        full((1, _H)),
        full((_H, _F)),
        full((1, _F)),
    ]
    out_specs = [
        pl.BlockSpec((_BN, _F), lambda i: (i, 0)),
        pl.BlockSpec((_BN, 3), lambda i: (i, 0)),
    ]
    out_shape = [
        jax.ShapeDtypeStruct((_N, _F), jnp.float32),
        jax.ShapeDtypeStruct((_N, 3), jnp.float32),
    ]
    return in_specs, out_specs, out_shape


def kernel(h, coord, edge_attr, edge_index, We1, be1, We2, be2, Wn1, bn1,
           Wn2, bn2, Wc1, bc1, Wc2, A, w):
    col = edge_index[1]
    hc, ccx, ccy, ccz = _get_sc_gather()(
        h, coord[:, 0], coord[:, 1], coord[:, 2], col)
    ncx = ccx.reshape(_N, _DEG)
    ncy = ccy.reshape(_N, _DEG)
    ncz = ccz.reshape(_N, _DEG)
    We1a = We1[0:_F]
    We1b = We1[_F:2 * _F]
    Wl = We1[2 * _F:2 * _F + 9]
    Wl6 = jnp.concatenate([
        Wl[0:1], Wl[1:2] + Wl[3:4], Wl[2:3] + Wl[6:7],
        Wl[4:5], Wl[5:6] + Wl[7:8], Wl[8:9],
        jnp.zeros((2, _H), jnp.float32)], axis=0)
    We1g = We1[2 * _F + 9:2 * _F + 9 + _H]
    We1e = We1[2 * _F + 9 + _H:]
    Apad = jnp.pad(A, ((0, 5), (0, 0)))
    in_specs, out_specs, out_shape = _tc_specs()
    h_out, coord_out = pl.pallas_call(
        _tc_body,
        grid=(_GRID,),
        in_specs=in_specs,
        out_specs=out_specs,
        out_shape=out_shape,
        compiler_params=pltpu.CompilerParams(
            dimension_semantics=("arbitrary",)),
    )(h, coord, ccx.reshape(_E, 1), ccy.reshape(_E, 1), ccz.reshape(_E, 1),
      ncx, ncy, ncz, hc, edge_attr,
      We1a, We1b, Wl6, We1g, We1e, be1[None], We2, be2[None],
      Wc1, bc1[None], Wc2.T, Apad, w[0], Wn1[:_F], Wn1[_F:],
      bn1[None], Wn2, bn2[None])
    return h_out, coord_out


# speedup vs baseline: 40.6690x; 1.0484x over previous
import functools
import jax
import jax.numpy as jnp
from jax import lax
from jax.experimental import pallas as pl
from jax.experimental.pallas import tpu as pltpu
from jax.experimental.pallas import tpu_sc as plsc

_N = 10000
_DEG = 16
_F = 128
_H = 128
_DE = 16
_E = _N * _DEG

_BN = 40
_BE = _BN * _DEG
_GRID = _N // _BN

_PAIRS = (
    (0, 1), (2, 3), (4, 5), (6, 7), (8, 9), (10, 11), (12, 13), (14, 15),
    (0, 2), (1, 3), (4, 6), (5, 7), (8, 10), (9, 11), (12, 14), (13, 15),
    (1, 2), (5, 6), (9, 10), (13, 14), (0, 4), (1, 5), (2, 6), (3, 7),
    (8, 12), (9, 13), (10, 14), (11, 15), (2, 4), (3, 5), (10, 12), (11, 13),
    (1, 2), (3, 4), (5, 6), (9, 10), (11, 12), (13, 14), (0, 8), (1, 9),
    (2, 10), (3, 11), (4, 12), (5, 13), (6, 14), (7, 15), (4, 8), (5, 9),
    (6, 10), (7, 11), (2, 4), (3, 5), (6, 8), (7, 9), (10, 12), (11, 13),
    (1, 2), (3, 4), (5, 6), (7, 8), (9, 10), (11, 12), (13, 14),
)


def _silu(x):
    return x * (1.0 / (1.0 + jnp.exp(-x)))


def _ssqrt(sq):
    pos = sq > 1e-12
    return jnp.where(pos, jnp.sqrt(jnp.where(pos, sq, 1.0)), 0.0)


def _dot(a, b):
    return lax.dot_general(a, b, (((1,), (0,)), ((), ())),
                           preferred_element_type=jnp.float32)


_NC = 2
_NS = 16
_NW = _NC * _NS
_CH = 128
_NCHUNK = _E // _CH
_MAXIT = (_NCHUNK + _NW - 1) // _NW

_sc_gather_built = None


def _get_sc_gather():
    global _sc_gather_built
    if _sc_gather_built is not None:
        return _sc_gather_built
    mesh = plsc.VectorSubcoreMesh(core_axis_name="c", subcore_axis_name="s")

    @functools.partial(
        pl.kernel,
        out_type=(jax.ShapeDtypeStruct((_E, _F), jnp.float32),
                  jax.ShapeDtypeStruct((_E,), jnp.float32),
                  jax.ShapeDtypeStruct((_E,), jnp.float32),
                  jax.ShapeDtypeStruct((_E,), jnp.float32)),
        mesh=mesh,
        scratch_types=[
            pltpu.VMEM((_CH,), jnp.int32),
            pltpu.VMEM((_CH, _F), jnp.float32),
            pltpu.VMEM((_CH,), jnp.float32),
            pltpu.VMEM((_CH,), jnp.float32),
            pltpu.VMEM((_CH,), jnp.float32),
            pltpu.VMEM((_N,), jnp.float32),
            pltpu.VMEM((_N,), jnp.float32),
            pltpu.VMEM((_N,), jnp.float32),
            pltpu.SemaphoreType.DMA,
        ],
        compiler_params=pltpu.CompilerParams(needs_layout_passes=False),
    )
    def _sc_gather(h_hbm, cx_hbm, cy_hbm, cz_hbm, col_hbm,
                   hc_out, ccx_out, ccy_out, ccz_out,
                   idx_v, hrows_v, gx_v, gy_v, gz_v, cx_v, cy_v, cz_v, s1):
        wid = lax.axis_index("s") * _NC + lax.axis_index("c")
        pltpu.sync_copy(cx_hbm, cx_v)
        pltpu.sync_copy(cy_hbm, cy_v)
        pltpu.sync_copy(cz_hbm, cz_v)

        def body(t, carry):
            cid = wid + t * _NW

            @pl.when(cid < _NCHUNK)
            def _():
                off = cid * _CH
                pltpu.sync_copy(col_hbm.at[pl.ds(off, _CH)], idx_v)
                cp1 = pltpu.async_copy(h_hbm.at[idx_v], hrows_v, s1)
                for j in range(_CH // 16):
                    sl = pl.ds(j * 16, 16)
                    ii = idx_v[sl]
                    gx_v[sl] = plsc.load_gather(cx_v, [ii])
                    gy_v[sl] = plsc.load_gather(cy_v, [ii])
                    gz_v[sl] = plsc.load_gather(cz_v, [ii])
                cp1.wait()
                pltpu.sync_copy(hrows_v, hc_out.at[pl.ds(off, _CH)])
                pltpu.sync_copy(gx_v, ccx_out.at[pl.ds(off, _CH)])
                pltpu.sync_copy(gy_v, ccy_out.at[pl.ds(off, _CH)])
                pltpu.sync_copy(gz_v, ccz_out.at[pl.ds(off, _CH)])

            return carry

        lax.fori_loop(0, _MAXIT, body, 0, unroll=False)

    _sc_gather_built = _sc_gather
    return _sc_gather_built


def _tc_body(h_ref, coord_ref, cjx_ref, cjy_ref, cjz_ref,
             ncx_ref, ncy_ref, ncz_ref, hc_ref,
             ea_ref, We1a, We1b, Wl6, We1g, We1e, b1, We2, b2, Wc1, bc1r,
             wc2r, WA, wT, Wn1a, Wn1b, bn1r, Wn2, bn2r,
             hout_ref, cout_ref, pb_ref, geo_ref):
    def bcast(a):
        bn, k = a.shape
        return jnp.broadcast_to(a[:, None, :], (bn, _DEG, k)).reshape(bn * _DEG, k)

    def d3(ax, ay, az, bx, by, bz):
        return _ssqrt((ax - bx) ** 2 + (ay - by) ** 2 + (az - bz) ** 2)

    hi = h_ref[...]
    cie = bcast(coord_ref[...])
    cix, ciy, ciz = cie[:, 0:1], cie[:, 1:2], cie[:, 2:3]
    cjx, cjy, cjz = cjx_ref[...], cjy_ref[...], cjz_ref[...]
    px = ciy * cjz - ciz * cjy
    py = ciz * cjx - cix * cjz
    pz = cix * cjy - ciy * cjx
    Nx = bcast(ncx_ref[...])
    Ny = bcast(ncy_ref[...])
    Nz = bcast(ncz_ref[...])
    N48x = jnp.concatenate([Nx, Nx, Nx], axis=1)
    N48y = jnp.concatenate([Ny, Ny, Ny], axis=1)
    N48z = jnp.concatenate([Nz, Nz, Nz], axis=1)
    bx16 = lambda v: jnp.broadcast_to(v, (_BE, _DEG))
    a48x = jnp.concatenate([bx16(px), bx16(cix), bx16(cjx)], axis=1)
    a48y = jnp.concatenate([bx16(py), bx16(ciy), bx16(cjy)], axis=1)
    a48z = jnp.concatenate([bx16(pz), bx16(ciz), bx16(cjz)], axis=1)
    X48 = d3(a48x, a48y, a48z, N48x, N48y, N48z)
    n_p = _ssqrt(px * px + py * py + pz * pz)
    n_i = _ssqrt(cix * cix + ciy * ciy + ciz * ciz)
    n_j = _ssqrt(cjx * cjx + cjy * cjy + cjz * cjz)
    d_pi = d3(px, py, pz, cix, ciy, ciz)
    d_pj = d3(px, py, pz, cjx, cjy, cjz)
    d_ij = d3(cix, ciy, ciz, cjx, cjy, cjz)
    L6 = jnp.concatenate([n_p, d_pi, d_pj, n_i, d_ij, n_j], axis=1)
    loc = _dot(L6, Wl6[...][0:6, :])
    pb_ref[...] = _dot(X48, WA[...])
    wTv = wT[...]

    def sort_chunk(i, carry):
        sl = pl.ds(i * 8, 8)
        vals = [pb_ref[sl, 128 * n:128 * (n + 1)] for n in range(_DEG)]
        for a, b in _PAIRS:
            lo = jnp.minimum(vals[a], vals[b])
            hi2 = jnp.maximum(vals[a], vals[b])
            vals[a] = lo
            vals[b] = hi2
        g = vals[0] * wTv[0:1, :]
        for n in range(1, _DEG):
            g = g + vals[n] * wTv[n:n + 1, :]
        geo_ref[sl, :] = g
        return carry

    lax.fori_loop(0, _BE // 8, sort_chunk, 0, unroll=False)
    geo = geo_ref[...]
    t1 = bcast(_dot(hi, We1a[...]))
    hid = _silu(t1 + _dot(hc_ref[...], We1b[...]) + loc
                + _dot(geo, We1g[...]) + _dot(ea_ref[...], We1e[...])
                + b1[...])
    ef = _silu(_dot(hid, We2[...]) + b2[...])
    u = _silu(_dot(ef, Wc1[...]) + bc1r[...])
    cs = jnp.sum(u * wc2r[...], axis=1, keepdims=True)
    tx = (cix - cjx) * cs
    ty = (ciy - cjy) * cs
    tz = (ciz - cjz) * cs
    inv = 1.0 / _DEG
    aggx = jnp.sum(tx.reshape(_BN, _DEG, 1), axis=1) * inv
    aggy = jnp.sum(ty.reshape(_BN, _DEG, 1), axis=1) * inv
    aggz = jnp.sum(tz.reshape(_BN, _DEG, 1), axis=1) * inv
    cout_ref[...] = coord_ref[...] + jnp.concatenate([aggx, aggy, aggz], axis=1)
    aggh = jnp.sum(ef.reshape(_BN, _DEG, _H), axis=1)
    nh = _silu(_dot(hi, Wn1a[...]) + _dot(aggh, Wn1b[...]) + bn1r[...])
    hout_ref[...] = hi + _dot(nh, Wn2[...]) + bn2r[...]


def _tc_specs():
    def nodes(k):
        return pl.BlockSpec((_BN, k), lambda i: (i, 0))

    def edges(k):
        return pl.BlockSpec((_BE, k), lambda i: (i, 0))

    def full(s):
        return pl.BlockSpec(s, lambda i: (0, 0))

    in_specs = [
        nodes(_F),
        nodes(3),
        edges(1),
        edges(1),
        edges(1),
        nodes(_DEG),
        nodes(_DEG),
        nodes(_DEG),
        edges(_F),
        edges(_DE),
        full((_F, _H)),
        full((_F, _H)),
        full((8, _H)),
        full((_H, _H)),
        full((_DE, _H)),
        full((1, _H)),
        full((_H, _H)),
        full((1, _H)),
        full((_H, _H)),
        full((1, _H)),
        full((1, _H)),
        full((48, _DEG * _H)),
        full((_DEG, _H)),
        full((_F, _H)),
        full((_F, _H)),
        full((1, _H)),
        full((_H, _F)),
        full((1, _F)),
    ]
    out_specs = [
        pl.BlockSpec((_BN, _F), lambda i: (i, 0)),
        pl.BlockSpec((_BN, 3), lambda i: (i, 0)),
    ]
    out_shape = [
        jax.ShapeDtypeStruct((_N, _F), jnp.float32),
        jax.ShapeDtypeStruct((_N, 3), jnp.float32),
    ]
    return in_specs, out_specs, out_shape


def kernel(h, coord, edge_attr, edge_index, We1, be1, We2, be2, Wn1, bn1,
           Wn2, bn2, Wc1, bc1, Wc2, A, w):
    col = edge_index[1]
    hc, ccx, ccy, ccz = _get_sc_gather()(
        h, coord[:, 0], coord[:, 1], coord[:, 2], col)
    ncx = ccx.reshape(_N, _DEG)
    ncy = ccy.reshape(_N, _DEG)
    ncz = ccz.reshape(_N, _DEG)
    We1a = We1[0:_F]
    We1b = We1[_F:2 * _F]
    Wl = We1[2 * _F:2 * _F + 9]
    Wl6 = jnp.concatenate([
        Wl[0:1], Wl[1:2] + Wl[3:4], Wl[2:3] + Wl[6:7],
        Wl[4:5], Wl[5:6] + Wl[7:8], Wl[8:9],
        jnp.zeros((2, _H), jnp.float32)], axis=0)
    We1g = We1[2 * _F + 9:2 * _F + 9 + _H]
    We1e = We1[2 * _F + 9 + _H:]
    eye16 = jnp.eye(_DEG, dtype=jnp.float32)
    WA = jnp.concatenate(
        [(eye16[:, :, None] * A[d][None, None, :]).reshape(_DEG, _DEG * _H)
         for d in range(3)], axis=0)
    in_specs, out_specs, out_shape = _tc_specs()
    h_out, coord_out = pl.pallas_call(
        _tc_body,
        grid=(_GRID,),
        in_specs=in_specs,
        out_specs=out_specs,
        out_shape=out_shape,
        compiler_params=pltpu.CompilerParams(
            dimension_semantics=("arbitrary",)),
        scratch_shapes=[
            pltpu.VMEM((_BE, _DEG * _H), jnp.float32),
            pltpu.VMEM((_BE, _H), jnp.float32),
        ],
    )(h, coord, ccx.reshape(_E, 1), ccy.reshape(_E, 1), ccz.reshape(_E, 1),
      ncx, ncy, ncz, hc, edge_attr,
      We1a, We1b, Wl6, We1g, We1e, be1[None], We2, be2[None],
      Wc1, bc1[None], Wc2.T, WA, w[0], Wn1[:_F], Wn1[_F:],
      bn1[None], Wn2, bn2[None])
    return h_out, coord_out


# speedup vs baseline: 44.6522x; 1.0979x over previous
import functools
import jax
import jax.numpy as jnp
from jax import lax
from jax.experimental import pallas as pl
from jax.experimental.pallas import tpu as pltpu
from jax.experimental.pallas import tpu_sc as plsc

_N = 10000
_DEG = 16
_F = 128
_H = 128
_DE = 16
_E = _N * _DEG

_BN = 80
_BE = _BN * _DEG
_GRID = _N // _BN

_PAIRS = (
    (0, 1), (2, 3), (4, 5), (6, 7), (8, 9), (10, 11), (12, 13), (14, 15),
    (0, 2), (1, 3), (4, 6), (5, 7), (8, 10), (9, 11), (12, 14), (13, 15),
    (1, 2), (5, 6), (9, 10), (13, 14), (0, 4), (1, 5), (2, 6), (3, 7),
    (8, 12), (9, 13), (10, 14), (11, 15), (2, 4), (3, 5), (10, 12), (11, 13),
    (1, 2), (3, 4), (5, 6), (9, 10), (11, 12), (13, 14), (0, 8), (1, 9),
    (2, 10), (3, 11), (4, 12), (5, 13), (6, 14), (7, 15), (4, 8), (5, 9),
    (6, 10), (7, 11), (2, 4), (3, 5), (6, 8), (7, 9), (10, 12), (11, 13),
    (1, 2), (3, 4), (5, 6), (7, 8), (9, 10), (11, 12), (13, 14),
)


def _silu(x):
    return x * (1.0 / (1.0 + jnp.exp(-x)))


def _ssqrt(sq):
    pos = sq > 1e-12
    return jnp.where(pos, jnp.sqrt(jnp.where(pos, sq, 1.0)), 0.0)


def _dot(a, b):
    return lax.dot_general(a, b, (((1,), (0,)), ((), ())),
                           preferred_element_type=jnp.float32)


_NC = 2
_NS = 16
_NW = _NC * _NS
_CH = 128
_NCHUNK = _E // _CH
_MAXIT = (_NCHUNK + _NW - 1) // _NW

_sc_gather_built = None


def _get_sc_gather():
    global _sc_gather_built
    if _sc_gather_built is not None:
        return _sc_gather_built
    mesh = plsc.VectorSubcoreMesh(core_axis_name="c", subcore_axis_name="s")

    @functools.partial(
        pl.kernel,
        out_type=(jax.ShapeDtypeStruct((_E, _F), jnp.float32),
                  jax.ShapeDtypeStruct((_E,), jnp.float32),
                  jax.ShapeDtypeStruct((_E,), jnp.float32),
                  jax.ShapeDtypeStruct((_E,), jnp.float32)),
        mesh=mesh,
        scratch_types=[
            pltpu.VMEM((_CH,), jnp.int32),
            pltpu.VMEM((_CH, _F), jnp.float32),
            pltpu.VMEM((_CH,), jnp.float32),
            pltpu.VMEM((_CH,), jnp.float32),
            pltpu.VMEM((_CH,), jnp.float32),
            pltpu.VMEM((_N,), jnp.float32),
            pltpu.VMEM((_N,), jnp.float32),
            pltpu.VMEM((_N,), jnp.float32),
            pltpu.SemaphoreType.DMA,
        ],
        compiler_params=pltpu.CompilerParams(needs_layout_passes=False),
    )
    def _sc_gather(h_hbm, cx_hbm, cy_hbm, cz_hbm, col_hbm,
                   hc_out, ccx_out, ccy_out, ccz_out,
                   idx_v, hrows_v, gx_v, gy_v, gz_v, cx_v, cy_v, cz_v, s1):
        wid = lax.axis_index("s") * _NC + lax.axis_index("c")
        pltpu.sync_copy(cx_hbm, cx_v)
        pltpu.sync_copy(cy_hbm, cy_v)
        pltpu.sync_copy(cz_hbm, cz_v)

        def body(t, carry):
            cid = wid + t * _NW

            @pl.when(cid < _NCHUNK)
            def _():
                off = cid * _CH
                pltpu.sync_copy(col_hbm.at[pl.ds(off, _CH)], idx_v)
                cp1 = pltpu.async_copy(h_hbm.at[idx_v], hrows_v, s1)
                for j in range(_CH // 16):
                    sl = pl.ds(j * 16, 16)
                    ii = idx_v[sl]
                    gx_v[sl] = plsc.load_gather(cx_v, [ii])
                    gy_v[sl] = plsc.load_gather(cy_v, [ii])
                    gz_v[sl] = plsc.load_gather(cz_v, [ii])
                cp1.wait()
                pltpu.sync_copy(hrows_v, hc_out.at[pl.ds(off, _CH)])
                pltpu.sync_copy(gx_v, ccx_out.at[pl.ds(off, _CH)])
                pltpu.sync_copy(gy_v, ccy_out.at[pl.ds(off, _CH)])
                pltpu.sync_copy(gz_v, ccz_out.at[pl.ds(off, _CH)])

            return carry

        lax.fori_loop(0, _MAXIT, body, 0, unroll=False)

    _sc_gather_built = _sc_gather
    return _sc_gather_built


def _tc_body(h_ref, coord_ref, cjx_ref, cjy_ref, cjz_ref,
             ncx_ref, ncy_ref, ncz_ref, hc_ref,
             ea_ref, We1a, We1b, We1g, We1e, b1, We2, b2, Wc1, bc1r,
             wc2r, WA, wT, Wn1a, Wn1b, bn1r, Wn2, bn2r,
             hout_ref, cout_ref, pb_ref, geo_ref):
    def bcast(a):
        bn, k = a.shape
        return jnp.broadcast_to(a[:, None, :], (bn, _DEG, k)).reshape(bn * _DEG, k)

    def d3(ax, ay, az, bx, by, bz):
        return _ssqrt((ax - bx) ** 2 + (ay - by) ** 2 + (az - bz) ** 2)

    hi = h_ref[...]
    cie = bcast(coord_ref[...])
    cix, ciy, ciz = cie[:, 0:1], cie[:, 1:2], cie[:, 2:3]
    cjx, cjy, cjz = cjx_ref[...], cjy_ref[...], cjz_ref[...]
    px = ciy * cjz - ciz * cjy
    py = ciz * cjx - cix * cjz
    pz = cix * cjy - ciy * cjx
    Nx = bcast(ncx_ref[...])
    Ny = bcast(ncy_ref[...])
    Nz = bcast(ncz_ref[...])
    z1 = jnp.zeros((_BE, 1), jnp.float32)
    bx16 = lambda v: jnp.broadcast_to(v, (_BE, _DEG))
    ax = jnp.concatenate([bx16(px), bx16(cix), bx16(cjx),
                          px, px, px, cix, cix, cjx], axis=1)
    ay = jnp.concatenate([bx16(py), bx16(ciy), bx16(cjy),
                          py, py, py, ciy, ciy, cjy], axis=1)
    az = jnp.concatenate([bx16(pz), bx16(ciz), bx16(cjz),
                          pz, pz, pz, ciz, ciz, cjz], axis=1)
    bx = jnp.concatenate([Nx, Nx, Nx, z1, cix, cjx, z1, cjx, z1], axis=1)
    by = jnp.concatenate([Ny, Ny, Ny, z1, ciy, cjy, z1, cjy, z1], axis=1)
    bz = jnp.concatenate([Nz, Nz, Nz, z1, ciz, cjz, z1, cjz, z1], axis=1)
    X54 = d3(ax, ay, az, bx, by, bz)
    pbv = _dot(X54, WA[...])
    loc = pbv[:, _DEG * _H:]
    pb_ref[...] = pbv
    wTv = wT[...]

    def sort_chunk(i, carry):
        sl = pl.ds(i * 16, 16)
        vals = [pb_ref[sl, 128 * n:128 * (n + 1)] for n in range(_DEG)]
        for a, b in _PAIRS:
            lo = jnp.minimum(vals[a], vals[b])
            hi2 = jnp.maximum(vals[a], vals[b])
            vals[a] = lo
            vals[b] = hi2
        g = vals[0] * wTv[0:1, :]
        for n in range(1, _DEG):
            g = g + vals[n] * wTv[n:n + 1, :]
        geo_ref[sl, :] = g
        return carry

    lax.fori_loop(0, _BE // 16, sort_chunk, 0, unroll=False)
    geo = geo_ref[...]
    t1 = bcast(_dot(hi, We1a[...]))
    hid = _silu(t1 + _dot(hc_ref[...], We1b[...]) + loc
                + _dot(geo, We1g[...]) + _dot(ea_ref[...], We1e[...])
                + b1[...])
    ef = _silu(_dot(hid, We2[...]) + b2[...])
    u = _silu(_dot(ef, Wc1[...]) + bc1r[...])
    cs = jnp.sum(u * wc2r[...], axis=1, keepdims=True)
    tx = (cix - cjx) * cs
    ty = (ciy - cjy) * cs
    tz = (ciz - cjz) * cs
    inv = 1.0 / _DEG
    aggx = jnp.sum(tx.reshape(_BN, _DEG, 1), axis=1) * inv
    aggy = jnp.sum(ty.reshape(_BN, _DEG, 1), axis=1) * inv
    aggz = jnp.sum(tz.reshape(_BN, _DEG, 1), axis=1) * inv
    cout_ref[...] = coord_ref[...] + jnp.concatenate([aggx, aggy, aggz], axis=1)
    aggh = jnp.sum(ef.reshape(_BN, _DEG, _H), axis=1)
    nh = _silu(_dot(hi, Wn1a[...]) + _dot(aggh, Wn1b[...]) + bn1r[...])
    hout_ref[...] = hi + _dot(nh, Wn2[...]) + bn2r[...]


def _tc_specs():
    def nodes(k):
        return pl.BlockSpec((_BN, k), lambda i: (i, 0))

    def edges(k):
        return pl.BlockSpec((_BE, k), lambda i: (i, 0))

    def full(s):
        return pl.BlockSpec(s, lambda i: (0, 0))

    in_specs = [
        nodes(_F),
        nodes(3),
        edges(1),
        edges(1),
        edges(1),
        nodes(_DEG),
        nodes(_DEG),
        nodes(_DEG),
        edges(_F),
        edges(_DE),
        full((_F, _H)),
        full((_F, _H)),
        full((_H, _H)),
        full((_DE, _H)),
        full((1, _H)),
        full((_H, _H)),
        full((1, _H)),
        full((_H, _H)),
        full((1, _H)),
        full((1, _H)),
        full((54, _DEG * _H + _H)),
        full((_DEG, _H)),
        full((_F, _H)),
        full((_F, _H)),
        full((1, _H)),
        full((_H, _F)),
        full((1, _F)),
    ]
    out_specs = [
        pl.BlockSpec((_BN, _F), lambda i: (i, 0)),
        pl.BlockSpec((_BN, 3), lambda i: (i, 0)),
    ]
    out_shape = [
        jax.ShapeDtypeStruct((_N, _F), jnp.float32),
        jax.ShapeDtypeStruct((_N, 3), jnp.float32),
    ]
    return in_specs, out_specs, out_shape


def kernel(h, coord, edge_attr, edge_index, We1, be1, We2, be2, Wn1, bn1,
           Wn2, bn2, Wc1, bc1, Wc2, A, w):
    col = edge_index[1]
    hc, ccx, ccy, ccz = _get_sc_gather()(
        h, coord[:, 0], coord[:, 1], coord[:, 2], col)
    ncx = ccx.reshape(_N, _DEG)
    ncy = ccy.reshape(_N, _DEG)
    ncz = ccz.reshape(_N, _DEG)
    We1a = We1[0:_F]
    We1b = We1[_F:2 * _F]
    Wl = We1[2 * _F:2 * _F + 9]
    Wl6 = jnp.concatenate([
        Wl[0:1], Wl[1:2] + Wl[3:4], Wl[2:3] + Wl[6:7],
        Wl[4:5], Wl[5:6] + Wl[7:8], Wl[8:9],
        jnp.zeros((2, _H), jnp.float32)], axis=0)
    We1g = We1[2 * _F + 9:2 * _F + 9 + _H]
    We1e = We1[2 * _F + 9 + _H:]
    eye16 = jnp.eye(_DEG, dtype=jnp.float32)
    WA48 = jnp.concatenate(
        [(eye16[:, :, None] * A[d][None, None, :]).reshape(_DEG, _DEG * _H)
         for d in range(3)], axis=0)
    WA = jnp.zeros((54, _DEG * _H + _H), jnp.float32)
    WA = WA.at[0:48, 0:_DEG * _H].set(WA48)
    WA = WA.at[48:54, _DEG * _H:].set(Wl6[0:6])
    in_specs, out_specs, out_shape = _tc_specs()
    h_out, coord_out = pl.pallas_call(
        _tc_body,
        grid=(_GRID,),
        in_specs=in_specs,
        out_specs=out_specs,
        out_shape=out_shape,
        compiler_params=pltpu.CompilerParams(
            dimension_semantics=("arbitrary",)),
        scratch_shapes=[
            pltpu.VMEM((_BE, _DEG * _H + _H), jnp.float32),
            pltpu.VMEM((_BE, _H), jnp.float32),
        ],
    )(h, coord, ccx.reshape(_E, 1), ccy.reshape(_E, 1), ccz.reshape(_E, 1),
      ncx, ncy, ncz, hc, edge_attr,
      We1a, We1b, We1g, We1e, be1[None], We2, be2[None],
      Wc1, bc1[None], Wc2.T, WA, w[0], Wn1[:_F], Wn1[_F:],
      bn1[None], Wn2, bn2[None])
    return h_out, coord_out


# speedup vs baseline: 68.6359x; 1.5371x over previous
import functools
import jax
import jax.numpy as jnp
import numpy as np
from jax import lax
from jax.experimental import pallas as pl
from jax.experimental.pallas import tpu as pltpu
from jax.experimental.pallas import tpu_sc as plsc

_N = 10000
_DEG = 16
_F = 128
_H = 128
_DE = 16
_E = _N * _DEG

_BN = 80
_BE = _BN * _DEG
_GRID = _N // _BN

_PAIRS = (
    (0, 1), (2, 3), (4, 5), (6, 7), (8, 9), (10, 11), (12, 13), (14, 15),
    (0, 2), (1, 3), (4, 6), (5, 7), (8, 10), (9, 11), (12, 14), (13, 15),
    (1, 2), (5, 6), (9, 10), (13, 14), (0, 4), (1, 5), (2, 6), (3, 7),
    (8, 12), (9, 13), (10, 14), (11, 15), (2, 4), (3, 5), (10, 12), (11, 13),
    (1, 2), (3, 4), (5, 6), (9, 10), (11, 12), (13, 14), (0, 8), (1, 9),
    (2, 10), (3, 11), (4, 12), (5, 13), (6, 14), (7, 15), (4, 8), (5, 9),
    (6, 10), (7, 11), (2, 4), (3, 5), (6, 8), (7, 9), (10, 12), (11, 13),
    (1, 2), (3, 4), (5, 6), (7, 8), (9, 10), (11, 12), (13, 14),
)


def _silu(x):
    return x * (1.0 / (1.0 + jnp.exp(-x)))


def _ssqrt(sq):
    pos = sq > 1e-12
    return jnp.where(pos, jnp.sqrt(jnp.where(pos, sq, 1.0)), 0.0)


def _dot(a, b):
    return lax.dot_general(a, b, (((1,), (0,)), ((), ())),
                           preferred_element_type=jnp.float32)


_NC = 2
_NS = 16
_NW = _NC * _NS
_CH = 128
_NCHUNK = _E // _CH
_MAXIT = (_NCHUNK + _NW - 1) // _NW

_sc_gather_built = None


def _get_sc_gather():
    global _sc_gather_built
    if _sc_gather_built is not None:
        return _sc_gather_built
    mesh = plsc.VectorSubcoreMesh(core_axis_name="c", subcore_axis_name="s")

    @functools.partial(
        pl.kernel,
        out_type=(jax.ShapeDtypeStruct((_E, _F), jnp.float32),
                  jax.ShapeDtypeStruct((_E,), jnp.float32),
                  jax.ShapeDtypeStruct((_E,), jnp.float32),
                  jax.ShapeDtypeStruct((_E,), jnp.float32)),
        mesh=mesh,
        scratch_types=[
            pltpu.VMEM((_CH,), jnp.int32),
            pltpu.VMEM((_CH, _F), jnp.float32),
            pltpu.VMEM((_CH,), jnp.float32),
            pltpu.VMEM((_CH,), jnp.float32),
            pltpu.VMEM((_CH,), jnp.float32),
            pltpu.VMEM((_N,), jnp.float32),
            pltpu.VMEM((_N,), jnp.float32),
            pltpu.VMEM((_N,), jnp.float32),
            pltpu.SemaphoreType.DMA,
        ],
        compiler_params=pltpu.CompilerParams(needs_layout_passes=False),
    )
    def _sc_gather(h_hbm, cx_hbm, cy_hbm, cz_hbm, col_hbm,
                   hc_out, ccx_out, ccy_out, ccz_out,
                   idx_v, hrows_v, gx_v, gy_v, gz_v, cx_v, cy_v, cz_v, s1):
        wid = lax.axis_index("s") * _NC + lax.axis_index("c")
        pltpu.sync_copy(cx_hbm, cx_v)
        pltpu.sync_copy(cy_hbm, cy_v)
        pltpu.sync_copy(cz_hbm, cz_v)

        def body(t, carry):
            cid = wid + t * _NW

            @pl.when(cid < _NCHUNK)
            def _():
                off = cid * _CH
                pltpu.sync_copy(col_hbm.at[pl.ds(off, _CH)], idx_v)
                cp1 = pltpu.async_copy(h_hbm.at[idx_v], hrows_v, s1)
                for j in range(_CH // 16):
                    sl = pl.ds(j * 16, 16)
                    ii = idx_v[sl]
                    gx_v[sl] = plsc.load_gather(cx_v, [ii])
                    gy_v[sl] = plsc.load_gather(cy_v, [ii])
                    gz_v[sl] = plsc.load_gather(cz_v, [ii])
                cp1.wait()
                pltpu.sync_copy(hrows_v, hc_out.at[pl.ds(off, _CH)])
                pltpu.sync_copy(gx_v, ccx_out.at[pl.ds(off, _CH)])
                pltpu.sync_copy(gy_v, ccy_out.at[pl.ds(off, _CH)])
                pltpu.sync_copy(gz_v, ccz_out.at[pl.ds(off, _CH)])

            return carry

        lax.fori_loop(0, _MAXIT, body, 0, unroll=False)

    _sc_gather_built = _sc_gather
    return _sc_gather_built


def _tc_body(h_ref, coord_ref, cje_ref,
             ncx_ref, ncy_ref, ncz_ref, hc_ref,
             ea_ref, We1a, We1b, We1g, We1e, b1, We2, b2, Wc1, bc1r,
             wc2r, WA, WDx, WDy, WDz, wT, Wn1a, Wn1b, bn1r, Wn2, bn2r,
             hout_ref, cout_ref, pb_ref, geo_ref):
    def bcast(a):
        bn, k = a.shape
        return jnp.broadcast_to(a[:, None, :], (bn, _DEG, k)).reshape(bn * _DEG, k)

    def roll3(v, s):
        return jnp.concatenate([v[:, s:3], v[:, 0:s]], axis=1)

    hi = h_ref[...]
    Ci3 = bcast(coord_ref[...])
    Cj3 = cje_ref[...]
    P3 = roll3(Ci3, 1) * roll3(Cj3, 2) - roll3(Ci3, 2) * roll3(Cj3, 1)
    Nx = bcast(ncx_ref[...])
    Ny = bcast(ncy_ref[...])
    Nz = bcast(ncz_ref[...])
    GG = jnp.concatenate([Nx, Ny, Nz, P3, Ci3, Cj3], axis=1)
    dx = _dot(GG, WDx[...])
    dy = _dot(GG, WDy[...])
    dz = _dot(GG, WDz[...])
    X54 = _ssqrt(dx * dx + dy * dy + dz * dz)
    pbv = _dot(X54, WA[...])
    loc = pbv[:, _DEG * _H:]
    pb_ref[...] = pbv
    wTv = wT[...]

    def sort_chunk(i, carry):
        sl = pl.ds(i * 16, 16)
        vals = [pb_ref[sl, 128 * n:128 * (n + 1)] for n in range(_DEG)]
        for a, b in _PAIRS:
            lo = jnp.minimum(vals[a], vals[b])
            hi2 = jnp.maximum(vals[a], vals[b])
            vals[a] = lo
            vals[b] = hi2
        g = vals[0] * wTv[0:1, :]
        for n in range(1, _DEG):
            g = g + vals[n] * wTv[n:n + 1, :]
        geo_ref[sl, :] = g
        return carry

    lax.fori_loop(0, _BE // 16, sort_chunk, 0, unroll=False)
    geo = geo_ref[...]
    t1 = bcast(_dot(hi, We1a[...]))
    hid = _silu(t1 + _dot(hc_ref[...], We1b[...]) + loc
                + _dot(geo, We1g[...]) + _dot(ea_ref[...], We1e[...])
                + b1[...])
    ef = _silu(_dot(hid, We2[...]) + b2[...])
    u = _silu(_dot(ef, Wc1[...]) + bc1r[...])
    cs = jnp.sum(u * wc2r[...], axis=1, keepdims=True)
    T3 = (Ci3 - Cj3) * cs
    agg3 = jnp.sum(T3.reshape(_BN, _DEG, 3), axis=1) * (1.0 / _DEG)
    cout_ref[...] = coord_ref[...] + agg3
    aggh = jnp.sum(ef.reshape(_BN, _DEG, _H), axis=1)
    nh = _silu(_dot(hi, Wn1a[...]) + _dot(aggh, Wn1b[...]) + bn1r[...])
    hout_ref[...] = hi + _dot(nh, Wn2[...]) + bn2r[...]


def _tc_specs():
    def nodes(k):
        return pl.BlockSpec((_BN, k), lambda i: (i, 0))

    def edges(k):
        return pl.BlockSpec((_BE, k), lambda i: (i, 0))

    def full(s):
        return pl.BlockSpec(s, lambda i: (0, 0))

    in_specs = [
        nodes(_F),
        nodes(3),
        edges(3),
        nodes(_DEG),
        nodes(_DEG),
        nodes(_DEG),
        edges(_F),
        edges(_DE),
        full((_F, _H)),
        full((_F, _H)),
        full((_H, _H)),
        full((_DE, _H)),
        full((1, _H)),
        full((_H, _H)),
        full((1, _H)),
        full((_H, _H)),
        full((1, _H)),
        full((1, _H)),
        full((54, _DEG * _H + _H)),
        full((57, 54)),
        full((57, 54)),
        full((57, 54)),
        full((_DEG, _H)),
        full((_F, _H)),
        full((_F, _H)),
        full((1, _H)),
        full((_H, _F)),
        full((1, _F)),
    ]
    out_specs = [
        pl.BlockSpec((_BN, _F), lambda i: (i, 0)),
        pl.BlockSpec((_BN, 3), lambda i: (i, 0)),
    ]
    out_shape = [
        jax.ShapeDtypeStruct((_N, _F), jnp.float32),
        jax.ShapeDtypeStruct((_N, 3), jnp.float32),
    ]
    return in_specs, out_specs, out_shape


def kernel(h, coord, edge_attr, edge_index, We1, be1, We2, be2, Wn1, bn1,
           Wn2, bn2, Wc1, bc1, Wc2, A, w):
    col = edge_index[1]
    hc, ccx, ccy, ccz = _get_sc_gather()(
        h, coord[:, 0], coord[:, 1], coord[:, 2], col)
    ncx = ccx.reshape(_N, _DEG)
    ncy = ccy.reshape(_N, _DEG)
    ncz = ccz.reshape(_N, _DEG)
    We1a = We1[0:_F]
    We1b = We1[_F:2 * _F]
    Wl = We1[2 * _F:2 * _F + 9]
    Wl6 = jnp.concatenate([
        Wl[0:1], Wl[1:2] + Wl[3:4], Wl[2:3] + Wl[6:7],
        Wl[4:5], Wl[5:6] + Wl[7:8], Wl[8:9],
        jnp.zeros((2, _H), jnp.float32)], axis=0)
    We1g = We1[2 * _F + 9:2 * _F + 9 + _H]
    We1e = We1[2 * _F + 9 + _H:]
    eye16 = jnp.eye(_DEG, dtype=jnp.float32)
    WA48 = jnp.concatenate(
        [(eye16[:, :, None] * A[d][None, None, :]).reshape(_DEG, _DEG * _H)
         for d in range(3)], axis=0)
    WA = jnp.zeros((54, _DEG * _H + _H), jnp.float32)
    WA = WA.at[0:48, 0:_DEG * _H].set(WA48)
    WA = WA.at[48:54, _DEG * _H:].set(Wl6[0:6])
    def build_wd(nrow0, pln, cil, cjl):
        M = np.zeros((57, 54), np.float32)
        for n in range(_DEG):
            M[nrow0 + n, n] = -1.0
            M[pln, n] += 1.0
            M[nrow0 + n, 16 + n] = -1.0
            M[cil, 16 + n] += 1.0
            M[nrow0 + n, 32 + n] = -1.0
            M[cjl, 32 + n] += 1.0
        M[pln, 48] += 1.0
        M[pln, 49] += 1.0
        M[cil, 49] -= 1.0
        M[pln, 50] += 1.0
        M[cjl, 50] -= 1.0
        M[cil, 51] += 1.0
        M[cil, 52] += 1.0
        M[cjl, 52] -= 1.0
        M[cjl, 53] += 1.0
        return jnp.asarray(M)

    WDx = build_wd(0, 48, 51, 54)
    WDy = build_wd(16, 49, 52, 55)
    WDz = build_wd(32, 50, 53, 56)
    in_specs, out_specs, out_shape = _tc_specs()
    h_out, coord_out = pl.pallas_call(
        _tc_body,
        grid=(_GRID,),
        in_specs=in_specs,
        out_specs=out_specs,
        out_shape=out_shape,
        compiler_params=pltpu.CompilerParams(
            dimension_semantics=("arbitrary",)),
        scratch_shapes=[
            pltpu.VMEM((_BE, _DEG * _H + _H), jnp.float32),
            pltpu.VMEM((_BE, _H), jnp.float32),
        ],
    )(h, coord, jnp.stack([ccx, ccy, ccz], axis=1),
      ncx, ncy, ncz, hc, edge_attr,
      We1a, We1b, We1g, We1e, be1[None], We2, be2[None],
      Wc1, bc1[None], Wc2.T, WA, WDx, WDy, WDz, w[0], Wn1[:_F], Wn1[_F:],
      bn1[None], Wn2, bn2[None])
    return h_out, coord_out


# speedup vs baseline: 73.4322x; 1.0699x over previous
import functools
import jax
import jax.numpy as jnp
import numpy as np
from jax import lax
from jax.experimental import pallas as pl
from jax.experimental.pallas import tpu as pltpu
from jax.experimental.pallas import tpu_sc as plsc

_N = 10000
_DEG = 16
_F = 128
_H = 128
_DE = 16
_E = _N * _DEG

_BN = 80
_BE = _BN * _DEG
_GRID = _N // _BN

_PAIRS = (
    (0, 1), (2, 3), (4, 5), (6, 7), (8, 9), (10, 11), (12, 13), (14, 15),
    (0, 2), (1, 3), (4, 6), (5, 7), (8, 10), (9, 11), (12, 14), (13, 15),
    (1, 2), (5, 6), (9, 10), (13, 14), (0, 4), (1, 5), (2, 6), (3, 7),
    (8, 12), (9, 13), (10, 14), (11, 15), (2, 4), (3, 5), (10, 12), (11, 13),
    (1, 2), (3, 4), (5, 6), (9, 10), (11, 12), (13, 14), (0, 8), (1, 9),
    (2, 10), (3, 11), (4, 12), (5, 13), (6, 14), (7, 15), (4, 8), (5, 9),
    (6, 10), (7, 11), (2, 4), (3, 5), (6, 8), (7, 9), (10, 12), (11, 13),
    (1, 2), (3, 4), (5, 6), (7, 8), (9, 10), (11, 12), (13, 14),
)


def _silu(x):
    return x * (1.0 / (1.0 + jnp.exp(-x)))


def _ssqrt(sq):
    pos = sq > 1e-12
    return jnp.where(pos, jnp.sqrt(jnp.where(pos, sq, 1.0)), 0.0)


def _dot(a, b):
    return lax.dot_general(a, b, (((1,), (0,)), ((), ())),
                           preferred_element_type=jnp.float32)


_NC = 2
_NS = 16
_NW = _NC * _NS
_CH = 128
_NCHUNK = _E // _CH
_MAXIT = (_NCHUNK + _NW - 1) // _NW

_sc_gather_built = None


def _get_sc_gather():
    global _sc_gather_built
    if _sc_gather_built is not None:
        return _sc_gather_built
    mesh = plsc.VectorSubcoreMesh(core_axis_name="c", subcore_axis_name="s")

    @functools.partial(
        pl.kernel,
        out_type=(jax.ShapeDtypeStruct((_E, _F), jnp.float32),
                  jax.ShapeDtypeStruct((_E,), jnp.float32),
                  jax.ShapeDtypeStruct((_E,), jnp.float32),
                  jax.ShapeDtypeStruct((_E,), jnp.float32)),
        mesh=mesh,
        scratch_types=[
            pltpu.VMEM((_CH,), jnp.int32),
            pltpu.VMEM((_CH, _F), jnp.float32),
            pltpu.VMEM((_CH,), jnp.float32),
            pltpu.VMEM((_CH,), jnp.float32),
            pltpu.VMEM((_CH,), jnp.float32),
            pltpu.VMEM((_N,), jnp.float32),
            pltpu.VMEM((_N,), jnp.float32),
            pltpu.VMEM((_N,), jnp.float32),
            pltpu.SemaphoreType.DMA,
        ],
        compiler_params=pltpu.CompilerParams(needs_layout_passes=False),
    )
    def _sc_gather(h_hbm, cx_hbm, cy_hbm, cz_hbm, col_hbm,
                   hc_out, ccx_out, ccy_out, ccz_out,
                   idx_v, hrows_v, gx_v, gy_v, gz_v, cx_v, cy_v, cz_v, s1):
        wid = lax.axis_index("s") * _NC + lax.axis_index("c")
        pltpu.sync_copy(cx_hbm, cx_v)
        pltpu.sync_copy(cy_hbm, cy_v)
        pltpu.sync_copy(cz_hbm, cz_v)

        def body(t, carry):
            cid = wid + t * _NW

            @pl.when(cid < _NCHUNK)
            def _():
                off = cid * _CH
                pltpu.sync_copy(col_hbm.at[pl.ds(off, _CH)], idx_v)
                cp1 = pltpu.async_copy(h_hbm.at[idx_v], hrows_v, s1)
                for j in range(_CH // 16):
                    sl = pl.ds(j * 16, 16)
                    ii = idx_v[sl]
                    gx_v[sl] = plsc.load_gather(cx_v, [ii])
                    gy_v[sl] = plsc.load_gather(cy_v, [ii])
                    gz_v[sl] = plsc.load_gather(cz_v, [ii])
                cp1.wait()
                pltpu.sync_copy(hrows_v, hc_out.at[pl.ds(off, _CH)])
                pltpu.sync_copy(gx_v, ccx_out.at[pl.ds(off, _CH)])
                pltpu.sync_copy(gy_v, ccy_out.at[pl.ds(off, _CH)])
                pltpu.sync_copy(gz_v, ccz_out.at[pl.ds(off, _CH)])

            return carry

        lax.fori_loop(0, _MAXIT, body, 0, unroll=False)

    _sc_gather_built = _sc_gather
    return _sc_gather_built


def _tc_body(h_ref, coord_ref, cje_ref,
             ncx_ref, ncy_ref, ncz_ref, hc_ref,
             ea_ref, We1a, We1b, We1g, We1e, b1, We2, b2, Wc1, bc1r,
             wc2r, WA, WDx, WDy, WDz, wT, Wn1a, Wn1b, bn1r, Wn2, bn2r,
             hout_ref, cout_ref, pb_ref, geo_ref):
    def bcast(a):
        bn, k = a.shape
        return jnp.broadcast_to(a[:, None, :], (bn, _DEG, k)).reshape(bn * _DEG, k)

    def roll3(v, s):
        return jnp.concatenate([v[:, s:3], v[:, 0:s]], axis=1)

    hi = h_ref[...]
    Ci3 = bcast(coord_ref[...])
    Cj3 = cje_ref[...]
    P3 = roll3(Ci3, 1) * roll3(Cj3, 2) - roll3(Ci3, 2) * roll3(Cj3, 1)
    Nx = bcast(ncx_ref[...])
    Ny = bcast(ncy_ref[...])
    Nz = bcast(ncz_ref[...])
    GG = jnp.concatenate([Nx, Ny, Nz, P3, Ci3, Cj3], axis=1)
    dx = _dot(GG, WDx[...])
    dy = _dot(GG, WDy[...])
    dz = _dot(GG, WDz[...])
    X54 = _ssqrt(dx * dx + dy * dy + dz * dz)
    pbv = _dot(X54, WA[...])
    loc = pbv[:, _DEG * _H:]
    pb_ref[...] = pbv
    wTv = wT[...]

    def sort_chunk(i, carry):
        sl = pl.ds(i * 16, 16)
        vals = [pb_ref[sl, 128 * n:128 * (n + 1)] for n in range(_DEG)]
        for a, b in _PAIRS:
            lo = jnp.minimum(vals[a], vals[b])
            hi2 = jnp.maximum(vals[a], vals[b])
            vals[a] = lo
            vals[b] = hi2
        g = vals[0] * wTv[0:1, :]
        for n in range(1, _DEG):
            g = g + vals[n] * wTv[n:n + 1, :]
        geo_ref[sl, :] = g
        return carry

    lax.fori_loop(0, _BE // 16, sort_chunk, 0, unroll=2)
    geo = geo_ref[...]
    t1 = bcast(_dot(hi, We1a[...]))
    hid = _silu(t1 + _dot(hc_ref[...], We1b[...]) + loc
                + _dot(geo, We1g[...]) + _dot(ea_ref[...], We1e[...])
                + b1[...])
    ef = _silu(_dot(hid, We2[...]) + b2[...])
    u = _silu(_dot(ef, Wc1[...]) + bc1r[...])
    cs = jnp.sum(u * wc2r[...], axis=1, keepdims=True)
    T3 = (Ci3 - Cj3) * cs
    agg3 = jnp.sum(T3.reshape(_BN, _DEG, 3), axis=1) * (1.0 / _DEG)
    cout_ref[...] = coord_ref[...] + agg3
    aggh = jnp.sum(ef.reshape(_BN, _DEG, _H), axis=1)
    nh = _silu(_dot(hi, Wn1a[...]) + _dot(aggh, Wn1b[...]) + bn1r[...])
    hout_ref[...] = hi + _dot(nh, Wn2[...]) + bn2r[...]


def _tc_specs():
    def nodes(k):
        return pl.BlockSpec((_BN, k), lambda i: (i, 0))

    def edges(k):
        return pl.BlockSpec((_BE, k), lambda i: (i, 0))

    def full(s):
        return pl.BlockSpec(s, lambda i: (0, 0))

    in_specs = [
        nodes(_F),
        nodes(3),
        edges(3),
        nodes(_DEG),
        nodes(_DEG),
        nodes(_DEG),
        edges(_F),
        edges(_DE),
        full((_F, _H)),
        full((_F, _H)),
        full((_H, _H)),
        full((_DE, _H)),
        full((1, _H)),
        full((_H, _H)),
        full((1, _H)),
        full((_H, _H)),
        full((1, _H)),
        full((1, _H)),
        full((54, _DEG * _H + _H)),
        full((57, 54)),
        full((57, 54)),
        full((57, 54)),
        full((_DEG, _H)),
        full((_F, _H)),
        full((_F, _H)),
        full((1, _H)),
        full((_H, _F)),
        full((1, _F)),
    ]
    out_specs = [
        pl.BlockSpec((_BN, _F), lambda i: (i, 0)),
        pl.BlockSpec((_BN, 3), lambda i: (i, 0)),
    ]
    out_shape = [
        jax.ShapeDtypeStruct((_N, _F), jnp.float32),
        jax.ShapeDtypeStruct((_N, 3), jnp.float32),
    ]
    return in_specs, out_specs, out_shape


def kernel(h, coord, edge_attr, edge_index, We1, be1, We2, be2, Wn1, bn1,
           Wn2, bn2, Wc1, bc1, Wc2, A, w):
    col = edge_index[1]
    hc, ccx, ccy, ccz = _get_sc_gather()(
        h, coord[:, 0], coord[:, 1], coord[:, 2], col)
    ncx = ccx.reshape(_N, _DEG)
    ncy = ccy.reshape(_N, _DEG)
    ncz = ccz.reshape(_N, _DEG)
    We1a = We1[0:_F]
    We1b = We1[_F:2 * _F]
    Wl = We1[2 * _F:2 * _F + 9]
    Wl6 = jnp.concatenate([
        Wl[0:1], Wl[1:2] + Wl[3:4], Wl[2:3] + Wl[6:7],
        Wl[4:5], Wl[5:6] + Wl[7:8], Wl[8:9],
        jnp.zeros((2, _H), jnp.float32)], axis=0)
    We1g = We1[2 * _F + 9:2 * _F + 9 + _H]
    We1e = We1[2 * _F + 9 + _H:]
    eye16 = jnp.eye(_DEG, dtype=jnp.float32)
    WA48 = jnp.concatenate(
        [(eye16[:, :, None] * A[d][None, None, :]).reshape(_DEG, _DEG * _H)
         for d in range(3)], axis=0)
    WA = jnp.zeros((54, _DEG * _H + _H), jnp.float32)
    WA = WA.at[0:48, 0:_DEG * _H].set(WA48)
    WA = WA.at[48:54, _DEG * _H:].set(Wl6[0:6])
    def build_wd(nrow0, pln, cil, cjl):
        M = np.zeros((57, 54), np.float32)
        for n in range(_DEG):
            M[nrow0 + n, n] = -1.0
            M[pln, n] += 1.0
            M[nrow0 + n, 16 + n] = -1.0
            M[cil, 16 + n] += 1.0
            M[nrow0 + n, 32 + n] = -1.0
            M[cjl, 32 + n] += 1.0
        M[pln, 48] += 1.0
        M[pln, 49] += 1.0
        M[cil, 49] -= 1.0
        M[pln, 50] += 1.0
        M[cjl, 50] -= 1.0
        M[cil, 51] += 1.0
        M[cil, 52] += 1.0
        M[cjl, 52] -= 1.0
        M[cjl, 53] += 1.0
        return jnp.asarray(M)

    WDx = build_wd(0, 48, 51, 54)
    WDy = build_wd(16, 49, 52, 55)
    WDz = build_wd(32, 50, 53, 56)
    in_specs, out_specs, out_shape = _tc_specs()
    h_out, coord_out = pl.pallas_call(
        _tc_body,
        grid=(_GRID,),
        in_specs=in_specs,
        out_specs=out_specs,
        out_shape=out_shape,
        compiler_params=pltpu.CompilerParams(
            dimension_semantics=("arbitrary",)),
        scratch_shapes=[
            pltpu.VMEM((_BE, _DEG * _H + _H), jnp.float32),
            pltpu.VMEM((_BE, _H), jnp.float32),
        ],
    )(h, coord, jnp.stack([ccx, ccy, ccz], axis=1),
      ncx, ncy, ncz, hc, edge_attr,
      We1a, We1b, We1g, We1e, be1[None], We2, be2[None],
      Wc1, bc1[None], Wc2.T, WA, WDx, WDy, WDz, w[0], Wn1[:_F], Wn1[_F:],
      bn1[None], Wn2, bn2[None])
    return h_out, coord_out


# speedup vs baseline: 75.8707x; 1.0332x over previous
import functools
import jax
import jax.numpy as jnp
import numpy as np
from jax import lax
from jax.experimental import pallas as pl
from jax.experimental.pallas import tpu as pltpu
from jax.experimental.pallas import tpu_sc as plsc

_N = 10000
_DEG = 16
_F = 128
_H = 128
_DE = 16
_E = _N * _DEG

_BN = 80
_BE = _BN * _DEG
_GRID = _N // _BN

_PAIRS = (
    (0, 1), (2, 3), (4, 5), (6, 7), (8, 9), (10, 11), (12, 13), (14, 15),
    (0, 2), (1, 3), (4, 6), (5, 7), (8, 10), (9, 11), (12, 14), (13, 15),
    (1, 2), (5, 6), (9, 10), (13, 14), (0, 4), (1, 5), (2, 6), (3, 7),
    (8, 12), (9, 13), (10, 14), (11, 15), (2, 4), (3, 5), (10, 12), (11, 13),
    (1, 2), (3, 4), (5, 6), (9, 10), (11, 12), (13, 14), (0, 8), (1, 9),
    (2, 10), (3, 11), (4, 12), (5, 13), (6, 14), (7, 15), (4, 8), (5, 9),
    (6, 10), (7, 11), (2, 4), (3, 5), (6, 8), (7, 9), (10, 12), (11, 13),
    (1, 2), (3, 4), (5, 6), (7, 8), (9, 10), (11, 12), (13, 14),
)


def _silu(x):
    return x * (1.0 / (1.0 + jnp.exp(-x)))


def _ssqrt(sq):
    pos = sq > 1e-12
    return jnp.where(pos, jnp.sqrt(jnp.where(pos, sq, 1.0)), 0.0)


def _dot(a, b):
    return lax.dot_general(a, b, (((1,), (0,)), ((), ())),
                           preferred_element_type=jnp.float32)


_NC = 2
_NS = 16
_NW = _NC * _NS
_CH = 128
_NCHUNK = _E // _CH
_MAXIT = (_NCHUNK + _NW - 1) // _NW

_sc_gather_built = None


def _get_sc_gather():
    global _sc_gather_built
    if _sc_gather_built is not None:
        return _sc_gather_built
    mesh = plsc.VectorSubcoreMesh(core_axis_name="c", subcore_axis_name="s")

    @functools.partial(
        pl.kernel,
        out_type=(jax.ShapeDtypeStruct((_E, _F), jnp.float32),
                  jax.ShapeDtypeStruct((_E,), jnp.float32),
                  jax.ShapeDtypeStruct((_E,), jnp.float32),
                  jax.ShapeDtypeStruct((_E,), jnp.float32)),
        mesh=mesh,
        scratch_types=[
            pltpu.VMEM((_CH,), jnp.int32),
            pltpu.VMEM((_CH, _F), jnp.float32),
            pltpu.VMEM((_CH,), jnp.float32),
            pltpu.VMEM((_CH,), jnp.float32),
            pltpu.VMEM((_CH,), jnp.float32),
            pltpu.VMEM((_N,), jnp.float32),
            pltpu.VMEM((_N,), jnp.float32),
            pltpu.VMEM((_N,), jnp.float32),
            pltpu.SemaphoreType.DMA,
        ],
        compiler_params=pltpu.CompilerParams(needs_layout_passes=False),
    )
    def _sc_gather(h_hbm, cx_hbm, cy_hbm, cz_hbm, col_hbm,
                   hc_out, ccx_out, ccy_out, ccz_out,
                   idx_v, hrows_v, gx_v, gy_v, gz_v, cx_v, cy_v, cz_v, s1):
        wid = lax.axis_index("s") * _NC + lax.axis_index("c")
        pltpu.sync_copy(cx_hbm, cx_v)
        pltpu.sync_copy(cy_hbm, cy_v)
        pltpu.sync_copy(cz_hbm, cz_v)

        def body(t, carry):
            cid = wid + t * _NW

            @pl.when(cid < _NCHUNK)
            def _():
                off = cid * _CH
                pltpu.sync_copy(col_hbm.at[pl.ds(off, _CH)], idx_v)
                cp1 = pltpu.async_copy(h_hbm.at[idx_v], hrows_v, s1)
                for j in range(_CH // 16):
                    sl = pl.ds(j * 16, 16)
                    ii = idx_v[sl]
                    gx_v[sl] = plsc.load_gather(cx_v, [ii])
                    gy_v[sl] = plsc.load_gather(cy_v, [ii])
                    gz_v[sl] = plsc.load_gather(cz_v, [ii])
                cp1.wait()
                pltpu.sync_copy(hrows_v, hc_out.at[pl.ds(off, _CH)])
                pltpu.sync_copy(gx_v, ccx_out.at[pl.ds(off, _CH)])
                pltpu.sync_copy(gy_v, ccy_out.at[pl.ds(off, _CH)])
                pltpu.sync_copy(gz_v, ccz_out.at[pl.ds(off, _CH)])

            return carry

        lax.fori_loop(0, _MAXIT, body, 0, unroll=False)

    _sc_gather_built = _sc_gather
    return _sc_gather_built


def _tc_body(h_ref, coord_ref, cje_ref,
             ncx_ref, ncy_ref, ncz_ref, hc_ref,
             ea_ref, We1a, We1b, We1g, We1e, b1, We2, b2, Wc1, bc1r,
             wc2r, WA, WDx, WDy, WDz, wT, Wn1a, Wn1b, bn1r, Wn2, bn2r,
             hout_ref, cout_ref, pb_ref, geo_ref):
    def bcast(a):
        bn, k = a.shape
        return jnp.broadcast_to(a[:, None, :], (bn, _DEG, k)).reshape(bn * _DEG, k)

    def roll3(v, s):
        return jnp.concatenate([v[:, s:3], v[:, 0:s]], axis=1)

    hi = h_ref[...]
    Ci3 = bcast(coord_ref[...])
    Cj3 = cje_ref[...]
    P3 = roll3(Ci3, 1) * roll3(Cj3, 2) - roll3(Ci3, 2) * roll3(Cj3, 1)
    Nx = bcast(ncx_ref[...])
    Ny = bcast(ncy_ref[...])
    Nz = bcast(ncz_ref[...])
    GG = jnp.concatenate([Nx, Ny, Nz, P3, Ci3, Cj3], axis=1)
    dx = _dot(GG, WDx[...])
    dy = _dot(GG, WDy[...])
    dz = _dot(GG, WDz[...])
    X54 = _ssqrt(dx * dx + dy * dy + dz * dz)
    pbv = _dot(X54, WA[...])
    loc = pbv[:, _DEG * _H:]
    pb_ref[...] = pbv
    wTv = wT[...]

    def sort_chunk(i, carry):
        sl = pl.ds(i * 16, 16)
        vals = [pb_ref[sl, 128 * n:128 * (n + 1)] for n in range(_DEG)]
        for a, b in _PAIRS:
            lo = jnp.minimum(vals[a], vals[b])
            hi2 = jnp.maximum(vals[a], vals[b])
            vals[a] = lo
            vals[b] = hi2
        g = vals[0] * wTv[0:1, :]
        for n in range(1, _DEG):
            g = g + vals[n] * wTv[n:n + 1, :]
        geo_ref[sl, :] = g
        return carry

    lax.fori_loop(0, _BE // 16, sort_chunk, 0, unroll=4)
    geo = geo_ref[...]
    t1 = bcast(_dot(hi, We1a[...]))
    hid = _silu(t1 + _dot(hc_ref[...], We1b[...]) + loc
                + _dot(geo, We1g[...]) + _dot(ea_ref[...], We1e[...])
                + b1[...])
    ef = _silu(_dot(hid, We2[...]) + b2[...])
    u = _silu(_dot(ef, Wc1[...]) + bc1r[...])
    cs = jnp.sum(u * wc2r[...], axis=1, keepdims=True)
    T3 = (Ci3 - Cj3) * cs
    agg3 = jnp.sum(T3.reshape(_BN, _DEG, 3), axis=1) * (1.0 / _DEG)
    cout_ref[...] = coord_ref[...] + agg3
    aggh = jnp.sum(ef.reshape(_BN, _DEG, _H), axis=1)
    nh = _silu(_dot(hi, Wn1a[...]) + _dot(aggh, Wn1b[...]) + bn1r[...])
    hout_ref[...] = hi + _dot(nh, Wn2[...]) + bn2r[...]


def _tc_specs():
    def nodes(k):
        return pl.BlockSpec((_BN, k), lambda i: (i, 0))

    def edges(k):
        return pl.BlockSpec((_BE, k), lambda i: (i, 0))

    def full(s):
        return pl.BlockSpec(s, lambda i: (0, 0))

    in_specs = [
        nodes(_F),
        nodes(3),
        edges(3),
        nodes(_DEG),
        nodes(_DEG),
        nodes(_DEG),
        edges(_F),
        edges(_DE),
        full((_F, _H)),
        full((_F, _H)),
        full((_H, _H)),
        full((_DE, _H)),
        full((1, _H)),
        full((_H, _H)),
        full((1, _H)),
        full((_H, _H)),
        full((1, _H)),
        full((1, _H)),
        full((54, _DEG * _H + _H)),
        full((57, 54)),
        full((57, 54)),
        full((57, 54)),
        full((_DEG, _H)),
        full((_F, _H)),
        full((_F, _H)),
        full((1, _H)),
        full((_H, _F)),
        full((1, _F)),
    ]
    out_specs = [
        pl.BlockSpec((_BN, _F), lambda i: (i, 0)),
        pl.BlockSpec((_BN, 3), lambda i: (i, 0)),
    ]
    out_shape = [
        jax.ShapeDtypeStruct((_N, _F), jnp.float32),
        jax.ShapeDtypeStruct((_N, 3), jnp.float32),
    ]
    return in_specs, out_specs, out_shape


def kernel(h, coord, edge_attr, edge_index, We1, be1, We2, be2, Wn1, bn1,
           Wn2, bn2, Wc1, bc1, Wc2, A, w):
    col = edge_index[1]
    hc, ccx, ccy, ccz = _get_sc_gather()(
        h, coord[:, 0], coord[:, 1], coord[:, 2], col)
    ncx = ccx.reshape(_N, _DEG)
    ncy = ccy.reshape(_N, _DEG)
    ncz = ccz.reshape(_N, _DEG)
    We1a = We1[0:_F]
    We1b = We1[_F:2 * _F]
    Wl = We1[2 * _F:2 * _F + 9]
    Wl6 = jnp.concatenate([
        Wl[0:1], Wl[1:2] + Wl[3:4], Wl[2:3] + Wl[6:7],
        Wl[4:5], Wl[5:6] + Wl[7:8], Wl[8:9],
        jnp.zeros((2, _H), jnp.float32)], axis=0)
    We1g = We1[2 * _F + 9:2 * _F + 9 + _H]
    We1e = We1[2 * _F + 9 + _H:]
    eye16 = jnp.eye(_DEG, dtype=jnp.float32)
    WA48 = jnp.concatenate(
        [(eye16[:, :, None] * A[d][None, None, :]).reshape(_DEG, _DEG * _H)
         for d in range(3)], axis=0)
    WA = jnp.zeros((54, _DEG * _H + _H), jnp.float32)
    WA = WA.at[0:48, 0:_DEG * _H].set(WA48)
    WA = WA.at[48:54, _DEG * _H:].set(Wl6[0:6])
    def build_wd(nrow0, pln, cil, cjl):
        M = np.zeros((57, 54), np.float32)
        for n in range(_DEG):
            M[nrow0 + n, n] = -1.0
            M[pln, n] += 1.0
            M[nrow0 + n, 16 + n] = -1.0
            M[cil, 16 + n] += 1.0
            M[nrow0 + n, 32 + n] = -1.0
            M[cjl, 32 + n] += 1.0
        M[pln, 48] += 1.0
        M[pln, 49] += 1.0
        M[cil, 49] -= 1.0
        M[pln, 50] += 1.0
        M[cjl, 50] -= 1.0
        M[cil, 51] += 1.0
        M[cil, 52] += 1.0
        M[cjl, 52] -= 1.0
        M[cjl, 53] += 1.0
        return jnp.asarray(M)

    WDx = build_wd(0, 48, 51, 54)
    WDy = build_wd(16, 49, 52, 55)
    WDz = build_wd(32, 50, 53, 56)
    in_specs, out_specs, out_shape = _tc_specs()
    h_out, coord_out = pl.pallas_call(
        _tc_body,
        grid=(_GRID,),
        in_specs=in_specs,
        out_specs=out_specs,
        out_shape=out_shape,
        compiler_params=pltpu.CompilerParams(
            dimension_semantics=("arbitrary",)),
        scratch_shapes=[
            pltpu.VMEM((_BE, _DEG * _H + _H), jnp.float32),
            pltpu.VMEM((_BE, _H), jnp.float32),
        ],
    )(h, coord, jnp.stack([ccx, ccy, ccz], axis=1),
      ncx, ncy, ncz, hc, edge_attr,
      We1a, We1b, We1g, We1e, be1[None], We2, be2[None],
      Wc1, bc1[None], Wc2.T, WA, WDx, WDy, WDz, w[0], Wn1[:_F], Wn1[_F:],
      bn1[None], Wn2, bn2[None])
    return h_out, coord_out


# speedup vs baseline: 77.3571x; 1.0196x over previous
import functools
import jax
import jax.numpy as jnp
import numpy as np
from jax import lax
from jax.experimental import pallas as pl
from jax.experimental.pallas import tpu as pltpu
from jax.experimental.pallas import tpu_sc as plsc

_N = 10000
_DEG = 16
_F = 128
_H = 128
_DE = 16
_E = _N * _DEG

_BN = 80
_BE = _BN * _DEG
_GRID = _N // _BN

_PAIRS = (
    (0, 1), (2, 3), (4, 5), (6, 7), (8, 9), (10, 11), (12, 13), (14, 15),
    (0, 2), (1, 3), (4, 6), (5, 7), (8, 10), (9, 11), (12, 14), (13, 15),
    (1, 2), (5, 6), (9, 10), (13, 14), (0, 4), (1, 5), (2, 6), (3, 7),
    (8, 12), (9, 13), (10, 14), (11, 15), (2, 4), (3, 5), (10, 12), (11, 13),
    (1, 2), (3, 4), (5, 6), (9, 10), (11, 12), (13, 14), (0, 8), (1, 9),
    (2, 10), (3, 11), (4, 12), (5, 13), (6, 14), (7, 15), (4, 8), (5, 9),
    (6, 10), (7, 11), (2, 4), (3, 5), (6, 8), (7, 9), (10, 12), (11, 13),
    (1, 2), (3, 4), (5, 6), (7, 8), (9, 10), (11, 12), (13, 14),
)


def _silu(x):
    return x * (1.0 / (1.0 + jnp.exp(-x)))


def _ssqrt(sq):
    pos = sq > 1e-12
    return jnp.where(pos, jnp.sqrt(jnp.where(pos, sq, 1.0)), 0.0)


def _dot(a, b):
    return lax.dot_general(a, b, (((1,), (0,)), ((), ())),
                           preferred_element_type=jnp.float32)


_NC = 2
_NS = 16
_NW = _NC * _NS
_CH = 128
_NCHUNK = _E // _CH
_MAXIT = (_NCHUNK + _NW - 1) // _NW

_sc_gather_built = None


def _get_sc_gather():
    global _sc_gather_built
    if _sc_gather_built is not None:
        return _sc_gather_built
    mesh = plsc.VectorSubcoreMesh(core_axis_name="c", subcore_axis_name="s")

    @functools.partial(
        pl.kernel,
        out_type=(jax.ShapeDtypeStruct((_E, _F), jnp.float32),
                  jax.ShapeDtypeStruct((_E,), jnp.float32),
                  jax.ShapeDtypeStruct((_E,), jnp.float32),
                  jax.ShapeDtypeStruct((_E,), jnp.float32)),
        mesh=mesh,
        scratch_types=[
            pltpu.VMEM((_CH,), jnp.int32),
            pltpu.VMEM((_CH, _F), jnp.float32),
            pltpu.VMEM((_CH,), jnp.float32),
            pltpu.VMEM((_CH,), jnp.float32),
            pltpu.VMEM((_CH,), jnp.float32),
            pltpu.VMEM((_N,), jnp.float32),
            pltpu.VMEM((_N,), jnp.float32),
            pltpu.VMEM((_N,), jnp.float32),
            pltpu.SemaphoreType.DMA,
        ],
        compiler_params=pltpu.CompilerParams(needs_layout_passes=False),
    )
    def _sc_gather(h_hbm, cx_hbm, cy_hbm, cz_hbm, col_hbm,
                   hc_out, ccx_out, ccy_out, ccz_out,
                   idx_v, hrows_v, gx_v, gy_v, gz_v, cx_v, cy_v, cz_v, s1):
        wid = lax.axis_index("s") * _NC + lax.axis_index("c")
        pltpu.sync_copy(cx_hbm, cx_v)
        pltpu.sync_copy(cy_hbm, cy_v)
        pltpu.sync_copy(cz_hbm, cz_v)

        def body(t, carry):
            cid = wid + t * _NW

            @pl.when(cid < _NCHUNK)
            def _():
                off = cid * _CH
                pltpu.sync_copy(col_hbm.at[pl.ds(off, _CH)], idx_v)
                cp1 = pltpu.async_copy(h_hbm.at[idx_v], hrows_v, s1)
                for j in range(_CH // 16):
                    sl = pl.ds(j * 16, 16)
                    ii = idx_v[sl]
                    gx_v[sl] = plsc.load_gather(cx_v, [ii])
                    gy_v[sl] = plsc.load_gather(cy_v, [ii])
                    gz_v[sl] = plsc.load_gather(cz_v, [ii])
                cp1.wait()
                pltpu.sync_copy(hrows_v, hc_out.at[pl.ds(off, _CH)])
                pltpu.sync_copy(gx_v, ccx_out.at[pl.ds(off, _CH)])
                pltpu.sync_copy(gy_v, ccy_out.at[pl.ds(off, _CH)])
                pltpu.sync_copy(gz_v, ccz_out.at[pl.ds(off, _CH)])

            return carry

        lax.fori_loop(0, _MAXIT, body, 0, unroll=False)

    _sc_gather_built = _sc_gather
    return _sc_gather_built


def _tc_body(h_ref, coord_ref, cje_ref,
             ncx_ref, ncy_ref, ncz_ref, hc_ref,
             ea_ref, We1a, We1b, We1g, We1e, b1, We2, b2, Wc1, bc1r,
             wc2c, WA, WD, wT, Wn1a, Wn1b, bn1r, Wn2, bn2r,
             hout_ref, cout_ref, pb_ref, geo_ref):
    def bcast(a):
        bn, k = a.shape
        return jnp.broadcast_to(a[:, None, :], (bn, _DEG, k)).reshape(bn * _DEG, k)

    def roll3(v, s):
        return jnp.concatenate([v[:, s:3], v[:, 0:s]], axis=1)

    hi = h_ref[...]
    Ci3 = bcast(coord_ref[...])
    Cj3 = cje_ref[...]
    P3 = roll3(Ci3, 1) * roll3(Cj3, 2) - roll3(Ci3, 2) * roll3(Cj3, 1)
    Nx = bcast(ncx_ref[...])
    Ny = bcast(ncy_ref[...])
    Nz = bcast(ncz_ref[...])
    GG = jnp.concatenate([Nx, Ny, Nz, P3, Ci3, Cj3], axis=1)
    D = _dot(GG, WD[...])
    dx = D[:, 0:64]
    dy = D[:, 64:128]
    dz = D[:, 128:192]
    X54 = _ssqrt(dx * dx + dy * dy + dz * dz)
    pbv = _dot(X54, WA[...])
    loc = pbv[:, _DEG * _H:]
    pb_ref[...] = pbv
    wTv = wT[...]

    def sort_chunk(i, carry):
        sl = pl.ds(i * 16, 16)
        vals = [pb_ref[sl, 128 * n:128 * (n + 1)] for n in range(_DEG)]
        for a, b in _PAIRS:
            lo = jnp.minimum(vals[a], vals[b])
            hi2 = jnp.maximum(vals[a], vals[b])
            vals[a] = lo
            vals[b] = hi2
        g = vals[0] * wTv[0:1, :]
        for n in range(1, _DEG):
            g = g + vals[n] * wTv[n:n + 1, :]
        geo_ref[sl, :] = g
        return carry

    lax.fori_loop(0, _BE // 16, sort_chunk, 0, unroll=8)
    geo = geo_ref[...]
    t1 = bcast(_dot(hi, We1a[...]))
    hid = _silu(t1 + _dot(hc_ref[...], We1b[...]) + loc
                + _dot(geo, We1g[...]) + _dot(ea_ref[...], We1e[...])
                + b1[...])
    ef = _silu(_dot(hid, We2[...]) + b2[...])
    u = _silu(_dot(ef, Wc1[...]) + bc1r[...])
    cs = _dot(u, wc2c[...])
    T3 = (Ci3 - Cj3) * cs
    agg3 = jnp.sum(T3.reshape(_BN, _DEG, 3), axis=1) * (1.0 / _DEG)
    cout_ref[...] = coord_ref[...] + agg3
    aggh = jnp.sum(ef.reshape(_BN, _DEG, _H), axis=1)
    nh = _silu(_dot(hi, Wn1a[...]) + _dot(aggh, Wn1b[...]) + bn1r[...])
    hout_ref[...] = hi + _dot(nh, Wn2[...]) + bn2r[...]


def _tc_specs():
    def nodes(k):
        return pl.BlockSpec((_BN, k), lambda i: (i, 0))

    def edges(k):
        return pl.BlockSpec((_BE, k), lambda i: (i, 0))

    def full(s):
        return pl.BlockSpec(s, lambda i: (0, 0))

    in_specs = [
        nodes(_F),
        nodes(3),
        edges(3),
        nodes(_DEG),
        nodes(_DEG),
        nodes(_DEG),
        edges(_F),
        edges(_DE),
        full((_F, _H)),
        full((_F, _H)),
        full((_H, _H)),
        full((_DE, _H)),
        full((1, _H)),
        full((_H, _H)),
        full((1, _H)),
        full((_H, _H)),
        full((1, _H)),
        full((_H, 1)),
        full((64, _DEG * _H + _H)),
        full((57, 192)),
        full((_DEG, _H)),
        full((_F, _H)),
        full((_F, _H)),
        full((1, _H)),
        full((_H, _F)),
        full((1, _F)),
    ]
    out_specs = [
        pl.BlockSpec((_BN, _F), lambda i: (i, 0)),
        pl.BlockSpec((_BN, 3), lambda i: (i, 0)),
    ]
    out_shape = [
        jax.ShapeDtypeStruct((_N, _F), jnp.float32),
        jax.ShapeDtypeStruct((_N, 3), jnp.float32),
    ]
    return in_specs, out_specs, out_shape


def kernel(h, coord, edge_attr, edge_index, We1, be1, We2, be2, Wn1, bn1,
           Wn2, bn2, Wc1, bc1, Wc2, A, w):
    col = edge_index[1]
    hc, ccx, ccy, ccz = _get_sc_gather()(
        h, coord[:, 0], coord[:, 1], coord[:, 2], col)
    ncx = ccx.reshape(_N, _DEG)
    ncy = ccy.reshape(_N, _DEG)
    ncz = ccz.reshape(_N, _DEG)
    We1a = We1[0:_F]
    We1b = We1[_F:2 * _F]
    Wl = We1[2 * _F:2 * _F + 9]
    Wl6 = jnp.concatenate([
        Wl[0:1], Wl[1:2] + Wl[3:4], Wl[2:3] + Wl[6:7],
        Wl[4:5], Wl[5:6] + Wl[7:8], Wl[8:9],
        jnp.zeros((2, _H), jnp.float32)], axis=0)
    We1g = We1[2 * _F + 9:2 * _F + 9 + _H]
    We1e = We1[2 * _F + 9 + _H:]
    eye16 = jnp.eye(_DEG, dtype=jnp.float32)
    WA48 = jnp.concatenate(
        [(eye16[:, :, None] * A[d][None, None, :]).reshape(_DEG, _DEG * _H)
         for d in range(3)], axis=0)
    WA = jnp.zeros((64, _DEG * _H + _H), jnp.float32)
    WA = WA.at[0:48, 0:_DEG * _H].set(WA48)
    WA = WA.at[48:54, _DEG * _H:].set(Wl6[0:6])
    def build_wd(nrow0, pln, cil, cjl):
        M = np.zeros((57, 54), np.float32)
        for n in range(_DEG):
            M[nrow0 + n, n] = -1.0
            M[pln, n] += 1.0
            M[nrow0 + n, 16 + n] = -1.0
            M[cil, 16 + n] += 1.0
            M[nrow0 + n, 32 + n] = -1.0
            M[cjl, 32 + n] += 1.0
        M[pln, 48] += 1.0
        M[pln, 49] += 1.0
        M[cil, 49] -= 1.0
        M[pln, 50] += 1.0
        M[cjl, 50] -= 1.0
        M[cil, 51] += 1.0
        M[cil, 52] += 1.0
        M[cjl, 52] -= 1.0
        M[cjl, 53] += 1.0
        return jnp.asarray(M)

    pad10 = ((0, 0), (0, 10))
    WD = jnp.concatenate([
        jnp.pad(build_wd(0, 48, 51, 54), pad10),
        jnp.pad(build_wd(16, 49, 52, 55), pad10),
        jnp.pad(build_wd(32, 50, 53, 56), pad10)], axis=1)
    in_specs, out_specs, out_shape = _tc_specs()
    h_out, coord_out = pl.pallas_call(
        _tc_body,
        grid=(_GRID,),
        in_specs=in_specs,
        out_specs=out_specs,
        out_shape=out_shape,
        compiler_params=pltpu.CompilerParams(
            dimension_semantics=("arbitrary",)),
        scratch_shapes=[
            pltpu.VMEM((_BE, _DEG * _H + _H), jnp.float32),
            pltpu.VMEM((_BE, _H), jnp.float32),
        ],
    )(h, coord, jnp.stack([ccx, ccy, ccz], axis=1),
      ncx, ncy, ncz, hc, edge_attr,
      We1a, We1b, We1g, We1e, be1[None], We2, be2[None],
      Wc1, bc1[None], Wc2, WA, WD, w[0], Wn1[:_F], Wn1[_F:],
      bn1[None], Wn2, bn2[None])
    return h_out, coord_out


# speedup vs baseline: 79.0079x; 1.0213x over previous
import functools
import jax
import jax.numpy as jnp
import numpy as np
from jax import lax
from jax.experimental import pallas as pl
from jax.experimental.pallas import tpu as pltpu
from jax.experimental.pallas import tpu_sc as plsc

_N = 10000
_DEG = 16
_F = 128
_H = 128
_DE = 16
_E = _N * _DEG

_BN = 80
_BE = _BN * _DEG
_GRID = _N // _BN

_PAIRS = (
    (0, 1), (2, 3), (4, 5), (6, 7), (8, 9), (10, 11), (12, 13), (14, 15),
    (0, 2), (1, 3), (4, 6), (5, 7), (8, 10), (9, 11), (12, 14), (13, 15),
    (1, 2), (5, 6), (9, 10), (13, 14), (0, 4), (1, 5), (2, 6), (3, 7),
    (8, 12), (9, 13), (10, 14), (11, 15), (2, 4), (3, 5), (10, 12), (11, 13),
    (1, 2), (3, 4), (5, 6), (9, 10), (11, 12), (13, 14), (0, 8), (1, 9),
    (2, 10), (3, 11), (4, 12), (5, 13), (6, 14), (7, 15), (4, 8), (5, 9),
    (6, 10), (7, 11), (2, 4), (3, 5), (6, 8), (7, 9), (10, 12), (11, 13),
    (1, 2), (3, 4), (5, 6), (7, 8), (9, 10), (11, 12), (13, 14),
)


def _silu(x):
    return x * (1.0 / (1.0 + jnp.exp(-x)))


def _ssqrt(sq):
    pos = sq > 1e-12
    return jnp.where(pos, jnp.sqrt(jnp.where(pos, sq, 1.0)), 0.0)


def _dot(a, b):
    return lax.dot_general(a, b, (((1,), (0,)), ((), ())),
                           preferred_element_type=jnp.float32)


_NC = 2
_NS = 16
_NW = _NC * _NS
_CH = 128
_NCHUNK = _E // _CH
_MAXIT = (_NCHUNK + _NW - 1) // _NW

_sc_gather_built = None


def _get_sc_gather():
    global _sc_gather_built
    if _sc_gather_built is not None:
        return _sc_gather_built
    mesh = plsc.VectorSubcoreMesh(core_axis_name="c", subcore_axis_name="s")

    @functools.partial(
        pl.kernel,
        out_type=(jax.ShapeDtypeStruct((_E, _F), jnp.float32),
                  jax.ShapeDtypeStruct((3, _E), jnp.float32)),
        mesh=mesh,
        scratch_types=[
            pltpu.VMEM((_CH,), jnp.int32),
            pltpu.VMEM((_CH,), jnp.int32),
            pltpu.VMEM((_CH, _F), jnp.float32),
            pltpu.VMEM((_CH, _F), jnp.float32),
            pltpu.VMEM((3, _CH), jnp.float32),
            pltpu.VMEM((3, _CH), jnp.float32),
            pltpu.VMEM((_N,), jnp.float32),
            pltpu.VMEM((_N,), jnp.float32),
            pltpu.VMEM((_N,), jnp.float32),
            pltpu.SemaphoreType.DMA,
            pltpu.SemaphoreType.DMA,
            pltpu.SemaphoreType.DMA,
            pltpu.SemaphoreType.DMA,
            pltpu.SemaphoreType.DMA,
            pltpu.SemaphoreType.DMA,
        ],
        compiler_params=pltpu.CompilerParams(needs_layout_passes=False),
    )
    def _sc_gather(h_hbm, cx_hbm, cy_hbm, cz_hbm, col_hbm,
                   hc_out, cc_out,
                   idx0, idx1, hr0, hr1, g0, g1, cx_v, cy_v, cz_v,
                   sh0, sh1, sw0, sw1, swg0, swg1):
        wid = lax.axis_index("s") * _NC + lax.axis_index("c")
        idx = (idx0, idx1)
        hr = (hr0, hr1)
        g = (g0, g1)
        sh = (sh0, sh1)
        sw = (sw0, sw1)
        swg = (swg0, swg1)
        pltpu.sync_copy(cx_hbm, cx_v)
        pltpu.sync_copy(cy_hbm, cy_v)
        pltpu.sync_copy(cz_hbm, cz_v)

        def pair(s, carry):
            for b in range(2):
                t = 2 * s + b
                cid = wid + t * _NW

                @pl.when(cid < _NCHUNK)
                def _():
                    off = cid * _CH
                    pltpu.sync_copy(col_hbm.at[pl.ds(off, _CH)], idx[b])

                    @pl.when(s >= 1)
                    def _():
                        pltpu.make_async_copy(
                            hr[b], hc_out.at[pl.ds(0, _CH)], sw[b]).wait()
                        pltpu.make_async_copy(
                            g[b], cc_out.at[:, pl.ds(0, _CH)], swg[b]).wait()

                    cph = pltpu.async_copy(h_hbm.at[idx[b]], hr[b], sh[b])
                    for j in range(_CH // 16):
                        sl = pl.ds(j * 16, 16)
                        ii = idx[b][sl]
                        g[b][0, sl] = plsc.load_gather(cx_v, [ii])
                        g[b][1, sl] = plsc.load_gather(cy_v, [ii])
                        g[b][2, sl] = plsc.load_gather(cz_v, [ii])
                    pltpu.async_copy(g[b], cc_out.at[:, pl.ds(off, _CH)],
                                     swg[b])
                    cph.wait()
                    pltpu.async_copy(hr[b], hc_out.at[pl.ds(off, _CH)], sw[b])

            return carry

        lax.fori_loop(0, _MAXIT // 2, pair, 0, unroll=False)
        for b in range(2):
            pltpu.make_async_copy(hr[b], hc_out.at[pl.ds(0, _CH)], sw[b]).wait()
            pltpu.make_async_copy(g[b], cc_out.at[:, pl.ds(0, _CH)],
                                  swg[b]).wait()

    _sc_gather_built = _sc_gather
    return _sc_gather_built


def _tc_body(h_ref, coord_ref, cje_ref,
             ncx_ref, ncy_ref, ncz_ref, hc_ref,
             ea_ref, We1a, We1b, We1g, We1e, b1, We2, b2, Wc1, bc1r,
             wc2c, WA, WD, wT, Wn1a, Wn1b, bn1r, Wn2, bn2r,
             hout_ref, cout_ref, pb_ref, geo_ref):
    def bcast(a):
        bn, k = a.shape
        return jnp.broadcast_to(a[:, None, :], (bn, _DEG, k)).reshape(bn * _DEG, k)

    def roll3(v, s):
        return jnp.concatenate([v[:, s:3], v[:, 0:s]], axis=1)

    hi = h_ref[...]
    Ci3 = bcast(coord_ref[...])
    Cj3 = cje_ref[...]
    P3 = roll3(Ci3, 1) * roll3(Cj3, 2) - roll3(Ci3, 2) * roll3(Cj3, 1)
    Nx = bcast(ncx_ref[...])
    Ny = bcast(ncy_ref[...])
    Nz = bcast(ncz_ref[...])
    GG = jnp.concatenate([Nx, Ny, Nz, P3, Ci3, Cj3], axis=1)
    D = _dot(GG, WD[...])
    dx = D[:, 0:64]
    dy = D[:, 64:128]
    dz = D[:, 128:192]
    X54 = _ssqrt(dx * dx + dy * dy + dz * dz)
    pbv = _dot(X54, WA[...])
    loc = pbv[:, _DEG * _H:]
    pb_ref[...] = pbv
    wTv = wT[...]

    def sort_chunk(i, carry):
        sl = pl.ds(i * 16, 16)
        vals = [pb_ref[sl, 128 * n:128 * (n + 1)] for n in range(_DEG)]
        for a, b in _PAIRS:
            lo = jnp.minimum(vals[a], vals[b])
            hi2 = jnp.maximum(vals[a], vals[b])
            vals[a] = lo
            vals[b] = hi2
        g = vals[0] * wTv[0:1, :]
        for n in range(1, _DEG):
            g = g + vals[n] * wTv[n:n + 1, :]
        geo_ref[sl, :] = g
        return carry

    lax.fori_loop(0, _BE // 16, sort_chunk, 0, unroll=8)
    geo = geo_ref[...]
    t1 = bcast(_dot(hi, We1a[...]))
    hid = _silu(t1 + _dot(hc_ref[...], We1b[...]) + loc
                + _dot(geo, We1g[...]) + _dot(ea_ref[...], We1e[...])
                + b1[...])
    ef = _silu(_dot(hid, We2[...]) + b2[...])
    u = _silu(_dot(ef, Wc1[...]) + bc1r[...])
    cs = _dot(u, wc2c[...])
    T3 = (Ci3 - Cj3) * cs
    agg3 = jnp.sum(T3.reshape(_BN, _DEG, 3), axis=1) * (1.0 / _DEG)
    cout_ref[...] = coord_ref[...] + agg3
    aggh = jnp.sum(ef.reshape(_BN, _DEG, _H), axis=1)
    nh = _silu(_dot(hi, Wn1a[...]) + _dot(aggh, Wn1b[...]) + bn1r[...])
    hout_ref[...] = hi + _dot(nh, Wn2[...]) + bn2r[...]


def _tc_specs():
    def nodes(k):
        return pl.BlockSpec((_BN, k), lambda i: (i, 0))

    def edges(k):
        return pl.BlockSpec((_BE, k), lambda i: (i, 0))

    def full(s):
        return pl.BlockSpec(s, lambda i: (0, 0))

    in_specs = [
        nodes(_F),
        nodes(3),
        edges(3),
        nodes(_DEG),
        nodes(_DEG),
        nodes(_DEG),
        edges(_F),
        edges(_DE),
        full((_F, _H)),
        full((_F, _H)),
        full((_H, _H)),
        full((_DE, _H)),
        full((1, _H)),
        full((_H, _H)),
        full((1, _H)),
        full((_H, _H)),
        full((1, _H)),
        full((_H, 1)),
        full((64, _DEG * _H + _H)),
        full((57, 192)),
        full((_DEG, _H)),
        full((_F, _H)),
        full((_F, _H)),
        full((1, _H)),
        full((_H, _F)),
        full((1, _F)),
    ]
    out_specs = [
        pl.BlockSpec((_BN, _F), lambda i: (i, 0)),
        pl.BlockSpec((_BN, 3), lambda i: (i, 0)),
    ]
    out_shape = [
        jax.ShapeDtypeStruct((_N, _F), jnp.float32),
        jax.ShapeDtypeStruct((_N, 3), jnp.float32),
    ]
    return in_specs, out_specs, out_shape


def kernel(h, coord, edge_attr, edge_index, We1, be1, We2, be2, Wn1, bn1,
           Wn2, bn2, Wc1, bc1, Wc2, A, w):
    col = edge_index[1]
    hc, cc = _get_sc_gather()(
        h, coord[:, 0], coord[:, 1], coord[:, 2], col)
    ncx = cc[0].reshape(_N, _DEG)
    ncy = cc[1].reshape(_N, _DEG)
    ncz = cc[2].reshape(_N, _DEG)
    We1a = We1[0:_F]
    We1b = We1[_F:2 * _F]
    Wl = We1[2 * _F:2 * _F + 9]
    Wl6 = jnp.concatenate([
        Wl[0:1], Wl[1:2] + Wl[3:4], Wl[2:3] + Wl[6:7],
        Wl[4:5], Wl[5:6] + Wl[7:8], Wl[8:9],
        jnp.zeros((2, _H), jnp.float32)], axis=0)
    We1g = We1[2 * _F + 9:2 * _F + 9 + _H]
    We1e = We1[2 * _F + 9 + _H:]
    eye16 = jnp.eye(_DEG, dtype=jnp.float32)
    WA48 = jnp.concatenate(
        [(eye16[:, :, None] * A[d][None, None, :]).reshape(_DEG, _DEG * _H)
         for d in range(3)], axis=0)
    WA = jnp.zeros((64, _DEG * _H + _H), jnp.float32)
    WA = WA.at[0:48, 0:_DEG * _H].set(WA48)
    WA = WA.at[48:54, _DEG * _H:].set(Wl6[0:6])
    def build_wd(nrow0, pln, cil, cjl):
        M = np.zeros((57, 54), np.float32)
        for n in range(_DEG):
            M[nrow0 + n, n] = -1.0
            M[pln, n] += 1.0
            M[nrow0 + n, 16 + n] = -1.0
            M[cil, 16 + n] += 1.0
            M[nrow0 + n, 32 + n] = -1.0
            M[cjl, 32 + n] += 1.0
        M[pln, 48] += 1.0
        M[pln, 49] += 1.0
        M[cil, 49] -= 1.0
        M[pln, 50] += 1.0
        M[cjl, 50] -= 1.0
        M[cil, 51] += 1.0
        M[cil, 52] += 1.0
        M[cjl, 52] -= 1.0
        M[cjl, 53] += 1.0
        return jnp.asarray(M)

    pad10 = ((0, 0), (0, 10))
    WD = jnp.concatenate([
        jnp.pad(build_wd(0, 48, 51, 54), pad10),
        jnp.pad(build_wd(16, 49, 52, 55), pad10),
        jnp.pad(build_wd(32, 50, 53, 56), pad10)], axis=1)
    in_specs, out_specs, out_shape = _tc_specs()
    h_out, coord_out = pl.pallas_call(
        _tc_body,
        grid=(_GRID,),
        in_specs=in_specs,
        out_specs=out_specs,
        out_shape=out_shape,
        compiler_params=pltpu.CompilerParams(
            dimension_semantics=("arbitrary",)),
        scratch_shapes=[
            pltpu.VMEM((_BE, _DEG * _H + _H), jnp.float32),
            pltpu.VMEM((_BE, _H), jnp.float32),
        ],
    )(h, coord, cc.T,
      ncx, ncy, ncz, hc, edge_attr,
      We1a, We1b, We1g, We1e, be1[None], We2, be2[None],
      Wc1, bc1[None], Wc2, WA, WD, w[0], Wn1[:_F], Wn1[_F:],
      bn1[None], Wn2, bn2[None])
    return h_out, coord_out


# speedup vs baseline: 81.8757x; 1.0363x over previous
import functools
import jax
import jax.numpy as jnp
import numpy as np
from jax import lax
from jax.experimental import pallas as pl
from jax.experimental.pallas import tpu as pltpu
from jax.experimental.pallas import tpu_sc as plsc

_N = 10000
_DEG = 16
_F = 128
_H = 128
_DE = 16
_E = _N * _DEG

_BN = 200
_BE = _BN * _DEG
_GRID = _N // _BN

_PAIRS = (
    (0, 1), (2, 3), (4, 5), (6, 7), (8, 9), (10, 11), (12, 13), (14, 15),
    (0, 2), (1, 3), (4, 6), (5, 7), (8, 10), (9, 11), (12, 14), (13, 15),
    (1, 2), (5, 6), (9, 10), (13, 14), (0, 4), (1, 5), (2, 6), (3, 7),
    (8, 12), (9, 13), (10, 14), (11, 15), (2, 4), (3, 5), (10, 12), (11, 13),
    (1, 2), (3, 4), (5, 6), (9, 10), (11, 12), (13, 14), (0, 8), (1, 9),
    (2, 10), (3, 11), (4, 12), (5, 13), (6, 14), (7, 15), (4, 8), (5, 9),
    (6, 10), (7, 11), (2, 4), (3, 5), (6, 8), (7, 9), (10, 12), (11, 13),
    (1, 2), (3, 4), (5, 6), (7, 8), (9, 10), (11, 12), (13, 14),
)


def _silu(x):
    return x * (1.0 / (1.0 + jnp.exp(-x)))


def _ssqrt(sq):
    pos = sq > 1e-12
    return jnp.where(pos, jnp.sqrt(jnp.where(pos, sq, 1.0)), 0.0)


def _dot(a, b):
    return lax.dot_general(a, b, (((1,), (0,)), ((), ())),
                           preferred_element_type=jnp.float32)


_NC = 2
_NS = 16
_NW = _NC * _NS
_CH = 128
_NCHUNK = _E // _CH
_MAXIT = (_NCHUNK + _NW - 1) // _NW

_sc_gather_built = None


def _get_sc_gather():
    global _sc_gather_built
    if _sc_gather_built is not None:
        return _sc_gather_built
    mesh = plsc.VectorSubcoreMesh(core_axis_name="c", subcore_axis_name="s")

    @functools.partial(
        pl.kernel,
        out_type=(jax.ShapeDtypeStruct((_E, _F), jnp.float32),
                  jax.ShapeDtypeStruct((3, _E), jnp.float32)),
        mesh=mesh,
        scratch_types=[
            pltpu.VMEM((_CH,), jnp.int32),
            pltpu.VMEM((_CH,), jnp.int32),
            pltpu.VMEM((_CH, _F), jnp.float32),
            pltpu.VMEM((_CH, _F), jnp.float32),
            pltpu.VMEM((3, _CH), jnp.float32),
            pltpu.VMEM((3, _CH), jnp.float32),
            pltpu.VMEM((_N,), jnp.float32),
            pltpu.VMEM((_N,), jnp.float32),
            pltpu.VMEM((_N,), jnp.float32),
            pltpu.SemaphoreType.DMA,
            pltpu.SemaphoreType.DMA,
            pltpu.SemaphoreType.DMA,
            pltpu.SemaphoreType.DMA,
            pltpu.SemaphoreType.DMA,
            pltpu.SemaphoreType.DMA,
        ],
        compiler_params=pltpu.CompilerParams(needs_layout_passes=False),
    )
    def _sc_gather(h_hbm, cx_hbm, cy_hbm, cz_hbm, col_hbm,
                   hc_out, cc_out,
                   idx0, idx1, hr0, hr1, g0, g1, cx_v, cy_v, cz_v,
                   sh0, sh1, sw0, sw1, swg0, swg1):
        wid = lax.axis_index("s") * _NC + lax.axis_index("c")
        idx = (idx0, idx1)
        hr = (hr0, hr1)
        g = (g0, g1)
        sh = (sh0, sh1)
        sw = (sw0, sw1)
        swg = (swg0, swg1)
        pltpu.sync_copy(cx_hbm, cx_v)
        pltpu.sync_copy(cy_hbm, cy_v)
        pltpu.sync_copy(cz_hbm, cz_v)

        def pair(s, carry):
            for b in range(2):
                t = 2 * s + b
                cid = wid + t * _NW

                @pl.when(cid < _NCHUNK)
                def _():
                    off = cid * _CH
                    pltpu.sync_copy(col_hbm.at[pl.ds(off, _CH)], idx[b])

                    @pl.when(s >= 1)
                    def _():
                        pltpu.make_async_copy(
                            hr[b], hc_out.at[pl.ds(0, _CH)], sw[b]).wait()
                        pltpu.make_async_copy(
                            g[b], cc_out.at[:, pl.ds(0, _CH)], swg[b]).wait()

                    cph = pltpu.async_copy(h_hbm.at[idx[b]], hr[b], sh[b])
                    for j in range(_CH // 16):
                        sl = pl.ds(j * 16, 16)
                        ii = idx[b][sl]
                        g[b][0, sl] = plsc.load_gather(cx_v, [ii])
                        g[b][1, sl] = plsc.load_gather(cy_v, [ii])
                        g[b][2, sl] = plsc.load_gather(cz_v, [ii])
                    pltpu.async_copy(g[b], cc_out.at[:, pl.ds(off, _CH)],
                                     swg[b])
                    cph.wait()
                    pltpu.async_copy(hr[b], hc_out.at[pl.ds(off, _CH)], sw[b])

            return carry

        lax.fori_loop(0, _MAXIT // 2, pair, 0, unroll=False)
        for b in range(2):
            pltpu.make_async_copy(hr[b], hc_out.at[pl.ds(0, _CH)], sw[b]).wait()
            pltpu.make_async_copy(g[b], cc_out.at[:, pl.ds(0, _CH)],
                                  swg[b]).wait()

    _sc_gather_built = _sc_gather
    return _sc_gather_built


def _tc_body(h_ref, coord_ref, cje_ref,
             ncx_ref, ncy_ref, ncz_ref, hc_ref,
             ea_ref, We1a, We1b, We1g, We1e, b1, We2, b2, Wc1, bc1r,
             wc2c, WA, WD, wT, Wn1a, Wn1b, bn1r, Wn2, bn2r,
             hout_ref, cout_ref, pb_ref, geo_ref):
    def bcast(a):
        bn, k = a.shape
        return jnp.broadcast_to(a[:, None, :], (bn, _DEG, k)).reshape(bn * _DEG, k)

    def roll3(v, s):
        return jnp.concatenate([v[:, s:3], v[:, 0:s]], axis=1)

    hi = h_ref[...]
    Ci3 = bcast(coord_ref[...])
    Cj3 = cje_ref[...]
    P3 = roll3(Ci3, 1) * roll3(Cj3, 2) - roll3(Ci3, 2) * roll3(Cj3, 1)
    Nx = bcast(ncx_ref[...])
    Ny = bcast(ncy_ref[...])
    Nz = bcast(ncz_ref[...])
    GG = jnp.concatenate([Nx, Ny, Nz, P3, Ci3, Cj3], axis=1)
    D = _dot(GG, WD[...])
    dx = D[:, 0:64]
    dy = D[:, 64:128]
    dz = D[:, 128:192]
    X54 = _ssqrt(dx * dx + dy * dy + dz * dz)
    pbv = _dot(X54, WA[...])
    loc = pbv[:, _DEG * _H:]
    pb_ref[...] = pbv
    wTv = wT[...]

    def sort_chunk(i, carry):
        sl = pl.ds(i * 16, 16)
        vals = [pb_ref[sl, 128 * n:128 * (n + 1)] for n in range(_DEG)]
        for a, b in _PAIRS:
            lo = jnp.minimum(vals[a], vals[b])
            hi2 = jnp.maximum(vals[a], vals[b])
            vals[a] = lo
            vals[b] = hi2
        g = vals[0] * wTv[0:1, :]
        for n in range(1, _DEG):
            g = g + vals[n] * wTv[n:n + 1, :]
        geo_ref[sl, :] = g
        return carry

    lax.fori_loop(0, _BE // 16, sort_chunk, 0, unroll=8)
    geo = geo_ref[...]
    t1 = bcast(_dot(hi, We1a[...]))
    hid = _silu(t1 + _dot(hc_ref[...], We1b[...]) + loc
                + _dot(geo, We1g[...]) + _dot(ea_ref[...], We1e[...])
                + b1[...])
    ef = _silu(_dot(hid, We2[...]) + b2[...])
    u = _silu(_dot(ef, Wc1[...]) + bc1r[...])
    cs = _dot(u, wc2c[...])
    T3 = (Ci3 - Cj3) * cs
    agg3 = jnp.sum(T3.reshape(_BN, _DEG, 3), axis=1) * (1.0 / _DEG)
    cout_ref[...] = coord_ref[...] + agg3
    aggh = jnp.sum(ef.reshape(_BN, _DEG, _H), axis=1)
    nh = _silu(_dot(hi, Wn1a[...]) + _dot(aggh, Wn1b[...]) + bn1r[...])
    hout_ref[...] = hi + _dot(nh, Wn2[...]) + bn2r[...]


def _tc_specs():
    def nodes(k):
        return pl.BlockSpec((_BN, k), lambda i: (i, 0))

    def edges(k):
        return pl.BlockSpec((_BE, k), lambda i: (i, 0))

    def full(s):
        return pl.BlockSpec(s, lambda i: (0, 0))

    in_specs = [
        nodes(_F),
        nodes(3),
        edges(3),
        nodes(_DEG),
        nodes(_DEG),
        nodes(_DEG),
        edges(_F),
        edges(_DE),
        full((_F, _H)),
        full((_F, _H)),
        full((_H, _H)),
        full((_DE, _H)),
        full((1, _H)),
        full((_H, _H)),
        full((1, _H)),
        full((_H, _H)),
        full((1, _H)),
        full((_H, 1)),
        full((64, _DEG * _H + _H)),
        full((57, 192)),
        full((_DEG, _H)),
        full((_F, _H)),
        full((_F, _H)),
        full((1, _H)),
        full((_H, _F)),
        full((1, _F)),
    ]
    out_specs = [
        pl.BlockSpec((_BN, _F), lambda i: (i, 0)),
        pl.BlockSpec((_BN, 3), lambda i: (i, 0)),
    ]
    out_shape = [
        jax.ShapeDtypeStruct((_N, _F), jnp.float32),
        jax.ShapeDtypeStruct((_N, 3), jnp.float32),
    ]
    return in_specs, out_specs, out_shape


def kernel(h, coord, edge_attr, edge_index, We1, be1, We2, be2, Wn1, bn1,
           Wn2, bn2, Wc1, bc1, Wc2, A, w):
    col = edge_index[1]
    hc, cc = _get_sc_gather()(
        h, coord[:, 0], coord[:, 1], coord[:, 2], col)
    ncx = cc[0].reshape(_N, _DEG)
    ncy = cc[1].reshape(_N, _DEG)
    ncz = cc[2].reshape(_N, _DEG)
    We1a = We1[0:_F]
    We1b = We1[_F:2 * _F]
    Wl = We1[2 * _F:2 * _F + 9]
    Wl6 = jnp.concatenate([
        Wl[0:1], Wl[1:2] + Wl[3:4], Wl[2:3] + Wl[6:7],
        Wl[4:5], Wl[5:6] + Wl[7:8], Wl[8:9],
        jnp.zeros((2, _H), jnp.float32)], axis=0)
    We1g = We1[2 * _F + 9:2 * _F + 9 + _H]
    We1e = We1[2 * _F + 9 + _H:]
    eye16 = jnp.eye(_DEG, dtype=jnp.float32)
    WA48 = jnp.concatenate(
        [(eye16[:, :, None] * A[d][None, None, :]).reshape(_DEG, _DEG * _H)
         for d in range(3)], axis=0)
    WA = jnp.zeros((64, _DEG * _H + _H), jnp.float32)
    WA = WA.at[0:48, 0:_DEG * _H].set(WA48)
    WA = WA.at[48:54, _DEG * _H:].set(Wl6[0:6])
    def build_wd(nrow0, pln, cil, cjl):
        M = np.zeros((57, 54), np.float32)
        for n in range(_DEG):
            M[nrow0 + n, n] = -1.0
            M[pln, n] += 1.0
            M[nrow0 + n, 16 + n] = -1.0
            M[cil, 16 + n] += 1.0
            M[nrow0 + n, 32 + n] = -1.0
            M[cjl, 32 + n] += 1.0
        M[pln, 48] += 1.0
        M[pln, 49] += 1.0
        M[cil, 49] -= 1.0
        M[pln, 50] += 1.0
        M[cjl, 50] -= 1.0
        M[cil, 51] += 1.0
        M[cil, 52] += 1.0
        M[cjl, 52] -= 1.0
        M[cjl, 53] += 1.0
        return jnp.asarray(M)

    pad10 = ((0, 0), (0, 10))
    WD = jnp.concatenate([
        jnp.pad(build_wd(0, 48, 51, 54), pad10),
        jnp.pad(build_wd(16, 49, 52, 55), pad10),
        jnp.pad(build_wd(32, 50, 53, 56), pad10)], axis=1)
    in_specs, out_specs, out_shape = _tc_specs()
    h_out, coord_out = pl.pallas_call(
        _tc_body,
        grid=(_GRID,),
        in_specs=in_specs,
        out_specs=out_specs,
        out_shape=out_shape,
        compiler_params=pltpu.CompilerParams(
            dimension_semantics=("arbitrary",)),
        scratch_shapes=[
            pltpu.VMEM((_BE, _DEG * _H + _H), jnp.float32),
            pltpu.VMEM((_BE, _H), jnp.float32),
        ],
    )(h, coord, cc.T,
      ncx, ncy, ncz, hc, edge_attr,
      We1a, We1b, We1g, We1e, be1[None], We2, be2[None],
      Wc1, bc1[None], Wc2, WA, WD, w[0], Wn1[:_F], Wn1[_F:],
      bn1[None], Wn2, bn2[None])
    return h_out, coord_out


# speedup vs baseline: 82.3759x; 1.0061x over previous
import functools
import jax
import jax.numpy as jnp
import numpy as np
from jax import lax
from jax.experimental import pallas as pl
from jax.experimental.pallas import tpu as pltpu
from jax.experimental.pallas import tpu_sc as plsc

_N = 10000
_DEG = 16
_F = 128
_H = 128
_DE = 16
_E = _N * _DEG

_BN = 200
_BE = _BN * _DEG
_GRID = _N // _BN

_PAIRS = (
    (0, 1), (2, 3), (4, 5), (6, 7), (8, 9), (10, 11), (12, 13), (14, 15),
    (0, 2), (1, 3), (4, 6), (5, 7), (8, 10), (9, 11), (12, 14), (13, 15),
    (1, 2), (5, 6), (9, 10), (13, 14), (0, 4), (1, 5), (2, 6), (3, 7),
    (8, 12), (9, 13), (10, 14), (11, 15), (2, 4), (3, 5), (10, 12), (11, 13),
    (1, 2), (3, 4), (5, 6), (9, 10), (11, 12), (13, 14), (0, 8), (1, 9),
    (2, 10), (3, 11), (4, 12), (5, 13), (6, 14), (7, 15), (4, 8), (5, 9),
    (6, 10), (7, 11), (2, 4), (3, 5), (6, 8), (7, 9), (10, 12), (11, 13),
    (1, 2), (3, 4), (5, 6), (7, 8), (9, 10), (11, 12), (13, 14),
)


def _silu(x):
    return x * (1.0 / (1.0 + jnp.exp(-x)))


def _ssqrt(sq):
    pos = sq > 1e-12
    return jnp.where(pos, jnp.sqrt(jnp.where(pos, sq, 1.0)), 0.0)


def _dot(a, b):
    return lax.dot_general(a, b, (((1,), (0,)), ((), ())),
                           preferred_element_type=jnp.float32)


_NC = 2
_NS = 16
_NW = _NC * _NS
_CH = 128
_NCHUNK = _E // _CH
_MAXIT = (_NCHUNK + _NW - 1) // _NW

_sc_gather_built = None


def _get_sc_gather():
    global _sc_gather_built
    if _sc_gather_built is not None:
        return _sc_gather_built
    mesh = plsc.VectorSubcoreMesh(core_axis_name="c", subcore_axis_name="s")

    @functools.partial(
        pl.kernel,
        out_type=(jax.ShapeDtypeStruct((_E, _F), jnp.float32),
                  jax.ShapeDtypeStruct((3, _E), jnp.float32)),
        mesh=mesh,
        scratch_types=[
            pltpu.VMEM((_CH,), jnp.int32),
            pltpu.VMEM((_CH,), jnp.int32),
            pltpu.VMEM((_CH, _F), jnp.float32),
            pltpu.VMEM((_CH, _F), jnp.float32),
            pltpu.VMEM((3, _CH), jnp.float32),
            pltpu.VMEM((3, _CH), jnp.float32),
            pltpu.VMEM((_N,), jnp.float32),
            pltpu.VMEM((_N,), jnp.float32),
            pltpu.VMEM((_N,), jnp.float32),
            pltpu.SemaphoreType.DMA,
            pltpu.SemaphoreType.DMA,
            pltpu.SemaphoreType.DMA,
            pltpu.SemaphoreType.DMA,
            pltpu.SemaphoreType.DMA,
            pltpu.SemaphoreType.DMA,
        ],
        compiler_params=pltpu.CompilerParams(needs_layout_passes=False),
    )
    def _sc_gather(h_hbm, cx_hbm, cy_hbm, cz_hbm, col_hbm,
                   hc_out, cc_out,
                   idx0, idx1, hr0, hr1, g0, g1, cx_v, cy_v, cz_v,
                   sh0, sh1, sw0, sw1, swg0, swg1):
        wid = lax.axis_index("s") * _NC + lax.axis_index("c")
        idx = (idx0, idx1)
        hr = (hr0, hr1)
        g = (g0, g1)
        sh = (sh0, sh1)
        sw = (sw0, sw1)
        swg = (swg0, swg1)
        pltpu.sync_copy(cx_hbm, cx_v)
        pltpu.sync_copy(cy_hbm, cy_v)
        pltpu.sync_copy(cz_hbm, cz_v)

        def pair(s, carry):
            for b in range(2):
                t = 2 * s + b
                cid = wid + t * _NW

                @pl.when(cid < _NCHUNK)
                def _():
                    off = cid * _CH
                    pltpu.sync_copy(col_hbm.at[pl.ds(off, _CH)], idx[b])

                    @pl.when(s >= 1)
                    def _():
                        pltpu.make_async_copy(
                            hr[b], hc_out.at[pl.ds(0, _CH)], sw[b]).wait()
                        pltpu.make_async_copy(
                            g[b], cc_out.at[:, pl.ds(0, _CH)], swg[b]).wait()

                    cph = pltpu.async_copy(h_hbm.at[idx[b]], hr[b], sh[b])
                    for j in range(_CH // 16):
                        sl = pl.ds(j * 16, 16)
                        ii = idx[b][sl]
                        g[b][0, sl] = plsc.load_gather(cx_v, [ii])
                        g[b][1, sl] = plsc.load_gather(cy_v, [ii])
                        g[b][2, sl] = plsc.load_gather(cz_v, [ii])
                    pltpu.async_copy(g[b], cc_out.at[:, pl.ds(off, _CH)],
                                     swg[b])
                    cph.wait()
                    pltpu.async_copy(hr[b], hc_out.at[pl.ds(off, _CH)], sw[b])

            return carry

        lax.fori_loop(0, _MAXIT // 2, pair, 0, unroll=False)
        for b in range(2):
            pltpu.make_async_copy(hr[b], hc_out.at[pl.ds(0, _CH)], sw[b]).wait()
            pltpu.make_async_copy(g[b], cc_out.at[:, pl.ds(0, _CH)],
                                  swg[b]).wait()

    _sc_gather_built = _sc_gather
    return _sc_gather_built


def _tc_body(h_ref, coord_ref, cje_ref,
             ncx_ref, ncy_ref, ncz_ref, hc_ref,
             ea_ref, We1a, We1b, We1g, We1e, b1, We2, b2, Wc1, bc1r,
             wc2c, WA, WD, wT, Wn1a, Wn1b, bn1r, Wn2, bn2r,
             hout_ref, cout_ref, pb_ref, geo_ref):
    def bcast(a):
        bn, k = a.shape
        return jnp.broadcast_to(a[:, None, :], (bn, _DEG, k)).reshape(bn * _DEG, k)

    def roll3(v, s):
        return jnp.concatenate([v[:, s:3], v[:, 0:s]], axis=1)

    hi = h_ref[...]
    Ci3 = bcast(coord_ref[...])
    Cj3 = cje_ref[...]
    P3 = roll3(Ci3, 1) * roll3(Cj3, 2) - roll3(Ci3, 2) * roll3(Cj3, 1)
    Nx = bcast(ncx_ref[...])
    Ny = bcast(ncy_ref[...])
    Nz = bcast(ncz_ref[...])
    GG = jnp.concatenate([Nx, Ny, Nz, P3, Ci3, Cj3], axis=1)
    D = _dot(GG, WD[...])
    dx = D[:, 0:64]
    dy = D[:, 64:128]
    dz = D[:, 128:192]
    X54 = _ssqrt(dx * dx + dy * dy + dz * dz)
    pbv = _dot(X54, WA[...])
    loc = pbv[:, _DEG * _H:]
    pb_ref[...] = pbv[:, 0:_DEG * _H]
    wTv = wT[...]

    def sort_chunk(i, carry):
        sl = pl.ds(i * 16, 16)
        vals = [pb_ref[sl, 128 * n:128 * (n + 1)] for n in range(_DEG)]
        for a, b in _PAIRS:
            lo = jnp.minimum(vals[a], vals[b])
            hi2 = jnp.maximum(vals[a], vals[b])
            vals[a] = lo
            vals[b] = hi2
        g = vals[0] * wTv[0:1, :]
        for n in range(1, _DEG):
            g = g + vals[n] * wTv[n:n + 1, :]
        geo_ref[sl, :] = g
        return carry

    lax.fori_loop(0, _BE // 16, sort_chunk, 0, unroll=8)
    geo = geo_ref[...]
    t1 = bcast(_dot(hi, We1a[...]))
    hid = _silu(t1 + _dot(hc_ref[...], We1b[...]) + loc
                + _dot(geo, We1g[...]) + _dot(ea_ref[...], We1e[...])
                + b1[...])
    ef = _silu(_dot(hid, We2[...]) + b2[...])
    u = _silu(_dot(ef, Wc1[...]) + bc1r[...])
    cs = _dot(u, wc2c[...])
    T3 = (Ci3 - Cj3) * cs
    agg3 = jnp.sum(T3.reshape(_BN, _DEG, 3), axis=1) * (1.0 / _DEG)
    cout_ref[...] = coord_ref[...] + agg3
    aggh = jnp.sum(ef.reshape(_BN, _DEG, _H), axis=1)
    nh = _silu(_dot(hi, Wn1a[...]) + _dot(aggh, Wn1b[...]) + bn1r[...])
    hout_ref[...] = hi + _dot(nh, Wn2[...]) + bn2r[...]


def _tc_specs():
    def nodes(k):
        return pl.BlockSpec((_BN, k), lambda i: (i, 0))

    def edges(k):
        return pl.BlockSpec((_BE, k), lambda i: (i, 0))

    def full(s):
        return pl.BlockSpec(s, lambda i: (0, 0))

    in_specs = [
        nodes(_F),
        nodes(3),
        edges(3),
        nodes(_DEG),
        nodes(_DEG),
        nodes(_DEG),
        edges(_F),
        edges(_DE),
        full((_F, _H)),
        full((_F, _H)),
        full((_H, _H)),
        full((_DE, _H)),
        full((1, _H)),
        full((_H, _H)),
        full((1, _H)),
        full((_H, _H)),
        full((1, _H)),
        full((_H, 1)),
        full((64, _DEG * _H + _H)),
        full((57, 192)),
        full((_DEG, _H)),
        full((_F, _H)),
        full((_F, _H)),
        full((1, _H)),
        full((_H, _F)),
        full((1, _F)),
    ]
    out_specs = [
        pl.BlockSpec((_BN, _F), lambda i: (i, 0)),
        pl.BlockSpec((_BN, 3), lambda i: (i, 0)),
    ]
    out_shape = [
        jax.ShapeDtypeStruct((_N, _F), jnp.float32),
        jax.ShapeDtypeStruct((_N, 3), jnp.float32),
    ]
    return in_specs, out_specs, out_shape


def kernel(h, coord, edge_attr, edge_index, We1, be1, We2, be2, Wn1, bn1,
           Wn2, bn2, Wc1, bc1, Wc2, A, w):
    col = edge_index[1]
    hc, cc = _get_sc_gather()(
        h, coord[:, 0], coord[:, 1], coord[:, 2], col)
    ncx = cc[0].reshape(_N, _DEG)
    ncy = cc[1].reshape(_N, _DEG)
    ncz = cc[2].reshape(_N, _DEG)
    We1a = We1[0:_F]
    We1b = We1[_F:2 * _F]
    Wl = We1[2 * _F:2 * _F + 9]
    Wl6 = jnp.concatenate([
        Wl[0:1], Wl[1:2] + Wl[3:4], Wl[2:3] + Wl[6:7],
        Wl[4:5], Wl[5:6] + Wl[7:8], Wl[8:9],
        jnp.zeros((2, _H), jnp.float32)], axis=0)
    We1g = We1[2 * _F + 9:2 * _F + 9 + _H]
    We1e = We1[2 * _F + 9 + _H:]
    eye16 = jnp.eye(_DEG, dtype=jnp.float32)
    WA48 = jnp.concatenate(
        [(eye16[:, :, None] * A[d][None, None, :]).reshape(_DEG, _DEG * _H)
         for d in range(3)], axis=0)
    WA = jnp.zeros((64, _DEG * _H + _H), jnp.float32)
    WA = WA.at[0:48, 0:_DEG * _H].set(WA48)
    WA = WA.at[48:54, _DEG * _H:].set(Wl6[0:6])
    def build_wd(nrow0, pln, cil, cjl):
        M = np.zeros((57, 54), np.float32)
        for n in range(_DEG):
            M[nrow0 + n, n] = -1.0
            M[pln, n] += 1.0
            M[nrow0 + n, 16 + n] = -1.0
            M[cil, 16 + n] += 1.0
            M[nrow0 + n, 32 + n] = -1.0
            M[cjl, 32 + n] += 1.0
        M[pln, 48] += 1.0
        M[pln, 49] += 1.0
        M[cil, 49] -= 1.0
        M[pln, 50] += 1.0
        M[cjl, 50] -= 1.0
        M[cil, 51] += 1.0
        M[cil, 52] += 1.0
        M[cjl, 52] -= 1.0
        M[cjl, 53] += 1.0
        return jnp.asarray(M)

    pad10 = ((0, 0), (0, 10))
    WD = jnp.concatenate([
        jnp.pad(build_wd(0, 48, 51, 54), pad10),
        jnp.pad(build_wd(16, 49, 52, 55), pad10),
        jnp.pad(build_wd(32, 50, 53, 56), pad10)], axis=1)
    in_specs, out_specs, out_shape = _tc_specs()
    h_out, coord_out = pl.pallas_call(
        _tc_body,
        grid=(_GRID,),
        in_specs=in_specs,
        out_specs=out_specs,
        out_shape=out_shape,
        compiler_params=pltpu.CompilerParams(
            dimension_semantics=("parallel",)),
        scratch_shapes=[
            pltpu.VMEM((_BE, _DEG * _H), jnp.float32),
            pltpu.VMEM((_BE, _H), jnp.float32),
        ],
    )(h, coord, cc.T,
      ncx, ncy, ncz, hc, edge_attr,
      We1a, We1b, We1g, We1e, be1[None], We2, be2[None],
      Wc1, bc1[None], Wc2, WA, WD, w[0], Wn1[:_F], Wn1[_F:],
      bn1[None], Wn2, bn2[None])
    return h_out, coord_out


# speedup vs baseline: 82.9423x; 1.0069x over previous
import functools
import jax
import jax.numpy as jnp
import numpy as np
from jax import lax
from jax.experimental import pallas as pl
from jax.experimental.pallas import tpu as pltpu
from jax.experimental.pallas import tpu_sc as plsc

_N = 10000
_DEG = 16
_F = 128
_H = 128
_DE = 16
_E = _N * _DEG

_BN = 200
_BE = _BN * _DEG
_GRID = _N // _BN

_PAIRS = (
    (0, 1), (2, 3), (4, 5), (6, 7), (8, 9), (10, 11), (12, 13), (14, 15),
    (0, 2), (1, 3), (4, 6), (5, 7), (8, 10), (9, 11), (12, 14), (13, 15),
    (1, 2), (5, 6), (9, 10), (13, 14), (0, 4), (1, 5), (2, 6), (3, 7),
    (8, 12), (9, 13), (10, 14), (11, 15), (2, 4), (3, 5), (10, 12), (11, 13),
    (1, 2), (3, 4), (5, 6), (9, 10), (11, 12), (13, 14), (0, 8), (1, 9),
    (2, 10), (3, 11), (4, 12), (5, 13), (6, 14), (7, 15), (4, 8), (5, 9),
    (6, 10), (7, 11), (2, 4), (3, 5), (6, 8), (7, 9), (10, 12), (11, 13),
    (1, 2), (3, 4), (5, 6), (7, 8), (9, 10), (11, 12), (13, 14),
)


def _silu(x):
    return x * (1.0 / (1.0 + jnp.exp(-x)))


def _ssqrt(sq):
    pos = sq > 1e-12
    return jnp.where(pos, jnp.sqrt(jnp.where(pos, sq, 1.0)), 0.0)


def _dot(a, b):
    return lax.dot_general(a, b, (((1,), (0,)), ((), ())),
                           preferred_element_type=jnp.float32)


_NC = 2
_NS = 16
_NW = _NC * _NS
_CH = 128
_NCHUNK = _E // _CH
_MAXIT = (_NCHUNK + _NW - 1) // _NW

_sc_gather_built = None


def _get_sc_gather():
    global _sc_gather_built
    if _sc_gather_built is not None:
        return _sc_gather_built
    mesh = plsc.VectorSubcoreMesh(core_axis_name="c", subcore_axis_name="s")

    @functools.partial(
        pl.kernel,
        out_type=(jax.ShapeDtypeStruct((_E, _F), jnp.float32),
                  jax.ShapeDtypeStruct((3, _E), jnp.float32)),
        mesh=mesh,
        scratch_types=[
            pltpu.VMEM((_CH,), jnp.int32),
            pltpu.VMEM((_CH,), jnp.int32),
            pltpu.VMEM((_CH, _F), jnp.float32),
            pltpu.VMEM((_CH, _F), jnp.float32),
            pltpu.VMEM((3, _CH), jnp.float32),
            pltpu.VMEM((3, _CH), jnp.float32),
            pltpu.VMEM((_N,), jnp.float32),
            pltpu.VMEM((_N,), jnp.float32),
            pltpu.VMEM((_N,), jnp.float32),
            pltpu.SemaphoreType.DMA,
            pltpu.SemaphoreType.DMA,
            pltpu.SemaphoreType.DMA,
            pltpu.SemaphoreType.DMA,
            pltpu.SemaphoreType.DMA,
            pltpu.SemaphoreType.DMA,
        ],
        compiler_params=pltpu.CompilerParams(needs_layout_passes=False),
    )
    def _sc_gather(h_hbm, cx_hbm, cy_hbm, cz_hbm, col_hbm,
                   hc_out, cc_out,
                   idx0, idx1, hr0, hr1, g0, g1, cx_v, cy_v, cz_v,
                   sh0, sh1, sw0, sw1, swg0, swg1):
        wid = lax.axis_index("s") * _NC + lax.axis_index("c")
        idx = (idx0, idx1)
        hr = (hr0, hr1)
        g = (g0, g1)
        sh = (sh0, sh1)
        sw = (sw0, sw1)
        swg = (swg0, swg1)
        pltpu.sync_copy(cx_hbm, cx_v)
        pltpu.sync_copy(cy_hbm, cy_v)
        pltpu.sync_copy(cz_hbm, cz_v)

        npairs = _NCHUNK // 2

        def pair(s, carry):
            p = wid + s * _NW

            @pl.when(p < npairs)
            def _():
                cph = []
                for b in range(2):
                    off = (2 * p + b) * _CH
                    pltpu.sync_copy(col_hbm.at[pl.ds(off, _CH)], idx[b])

                    @pl.when(s >= 1)
                    def _():
                        pltpu.make_async_copy(
                            hr[b], hc_out.at[pl.ds(0, _CH)], sw[b]).wait()
                        pltpu.make_async_copy(
                            g[b], cc_out.at[:, pl.ds(0, _CH)], swg[b]).wait()

                    cph.append(pltpu.async_copy(h_hbm.at[idx[b]], hr[b],
                                                sh[b]))
                for b in range(2):
                    off = (2 * p + b) * _CH
                    for j in range(_CH // 16):
                        sl = pl.ds(j * 16, 16)
                        ii = idx[b][sl]
                        g[b][0, sl] = plsc.load_gather(cx_v, [ii])
                        g[b][1, sl] = plsc.load_gather(cy_v, [ii])
                        g[b][2, sl] = plsc.load_gather(cz_v, [ii])
                    pltpu.async_copy(g[b], cc_out.at[:, pl.ds(off, _CH)],
                                     swg[b])
                for b in range(2):
                    off = (2 * p + b) * _CH
                    cph[b].wait()
                    pltpu.async_copy(hr[b], hc_out.at[pl.ds(off, _CH)], sw[b])

            return carry

        lax.fori_loop(0, (npairs + _NW - 1) // _NW, pair, 0, unroll=False)
        for b in range(2):
            pltpu.make_async_copy(hr[b], hc_out.at[pl.ds(0, _CH)], sw[b]).wait()
            pltpu.make_async_copy(g[b], cc_out.at[:, pl.ds(0, _CH)],
                                  swg[b]).wait()

    _sc_gather_built = _sc_gather
    return _sc_gather_built


def _tc_body(h_ref, coord_ref, cje_ref,
             ncx_ref, ncy_ref, ncz_ref, hc_ref,
             ea_ref, We1a, We1b, We1g, We1e, b1, We2, b2, Wc1, bc1r,
             wc2c, WA, WD, wT, Wn1a, Wn1b, bn1r, Wn2, bn2r,
             hout_ref, cout_ref, pb_ref, geo_ref):
    def bcast(a):
        bn, k = a.shape
        return jnp.broadcast_to(a[:, None, :], (bn, _DEG, k)).reshape(bn * _DEG, k)

    def roll3(v, s):
        return jnp.concatenate([v[:, s:3], v[:, 0:s]], axis=1)

    hi = h_ref[...]
    Ci3 = bcast(coord_ref[...])
    Cj3 = cje_ref[...]
    P3 = roll3(Ci3, 1) * roll3(Cj3, 2) - roll3(Ci3, 2) * roll3(Cj3, 1)
    Nx = bcast(ncx_ref[...])
    Ny = bcast(ncy_ref[...])
    Nz = bcast(ncz_ref[...])
    GG = jnp.concatenate([Nx, Ny, Nz, P3, Ci3, Cj3], axis=1)
    D = _dot(GG, WD[...])
    dx = D[:, 0:64]
    dy = D[:, 64:128]
    dz = D[:, 128:192]
    X54 = _ssqrt(dx * dx + dy * dy + dz * dz)
    pbv = _dot(X54, WA[...])
    loc = pbv[:, _DEG * _H:]
    pb_ref[...] = pbv[:, 0:_DEG * _H]
    wTv = wT[...]

    def sort_chunk(i, carry):
        sl = pl.ds(i * 16, 16)
        vals = [pb_ref[sl, 128 * n:128 * (n + 1)] for n in range(_DEG)]
        for a, b in _PAIRS:
            lo = jnp.minimum(vals[a], vals[b])
            hi2 = jnp.maximum(vals[a], vals[b])
            vals[a] = lo
            vals[b] = hi2
        g = vals[0] * wTv[0:1, :]
        for n in range(1, _DEG):
            g = g + vals[n] * wTv[n:n + 1, :]
        geo_ref[sl, :] = g
        return carry

    lax.fori_loop(0, _BE // 16, sort_chunk, 0, unroll=8)
    geo = geo_ref[...]
    t1 = bcast(_dot(hi, We1a[...]))
    hid = _silu(t1 + _dot(hc_ref[...], We1b[...]) + loc
                + _dot(geo, We1g[...]) + _dot(ea_ref[...], We1e[...])
                + b1[...])
    ef = _silu(_dot(hid, We2[...]) + b2[...])
    u = _silu(_dot(ef, Wc1[...]) + bc1r[...])
    cs = _dot(u, wc2c[...])
    T3 = (Ci3 - Cj3) * cs
    agg3 = jnp.sum(T3.reshape(_BN, _DEG, 3), axis=1) * (1.0 / _DEG)
    cout_ref[...] = coord_ref[...] + agg3
    aggh = jnp.sum(ef.reshape(_BN, _DEG, _H), axis=1)
    nh = _silu(_dot(hi, Wn1a[...]) + _dot(aggh, Wn1b[...]) + bn1r[...])
    hout_ref[...] = hi + _dot(nh, Wn2[...]) + bn2r[...]


def _tc_specs():
    def nodes(k):
        return pl.BlockSpec((_BN, k), lambda i: (i, 0))

    def edges(k):
        return pl.BlockSpec((_BE, k), lambda i: (i, 0))

    def full(s):
        return pl.BlockSpec(s, lambda i: (0, 0))

    in_specs = [
        nodes(_F),
        nodes(3),
        edges(3),
        nodes(_DEG),
        nodes(_DEG),
        nodes(_DEG),
        edges(_F),
        edges(_DE),
        full((_F, _H)),
        full((_F, _H)),
        full((_H, _H)),
        full((_DE, _H)),
        full((1, _H)),
        full((_H, _H)),
        full((1, _H)),
        full((_H, _H)),
        full((1, _H)),
        full((_H, 1)),
        full((64, _DEG * _H + _H)),
        full((57, 192)),
        full((_DEG, _H)),
        full((_F, _H)),
        full((_F, _H)),
        full((1, _H)),
        full((_H, _F)),
        full((1, _F)),
    ]
    out_specs = [
        pl.BlockSpec((_BN, _F), lambda i: (i, 0)),
        pl.BlockSpec((_BN, 3), lambda i: (i, 0)),
    ]
    out_shape = [
        jax.ShapeDtypeStruct((_N, _F), jnp.float32),
        jax.ShapeDtypeStruct((_N, 3), jnp.float32),
    ]
    return in_specs, out_specs, out_shape


def kernel(h, coord, edge_attr, edge_index, We1, be1, We2, be2, Wn1, bn1,
           Wn2, bn2, Wc1, bc1, Wc2, A, w):
    col = edge_index[1]
    hc, cc = _get_sc_gather()(
        h, coord[:, 0], coord[:, 1], coord[:, 2], col)
    ncx = cc[0].reshape(_N, _DEG)
    ncy = cc[1].reshape(_N, _DEG)
    ncz = cc[2].reshape(_N, _DEG)
    We1a = We1[0:_F]
    We1b = We1[_F:2 * _F]
    Wl = We1[2 * _F:2 * _F + 9]
    Wl6 = jnp.concatenate([
        Wl[0:1], Wl[1:2] + Wl[3:4], Wl[2:3] + Wl[6:7],
        Wl[4:5], Wl[5:6] + Wl[7:8], Wl[8:9],
        jnp.zeros((2, _H), jnp.float32)], axis=0)
    We1g = We1[2 * _F + 9:2 * _F + 9 + _H]
    We1e = We1[2 * _F + 9 + _H:]
    eye16 = jnp.eye(_DEG, dtype=jnp.float32)
    WA48 = jnp.concatenate(
        [(eye16[:, :, None] * A[d][None, None, :]).reshape(_DEG, _DEG * _H)
         for d in range(3)], axis=0)
    WA = jnp.zeros((64, _DEG * _H + _H), jnp.float32)
    WA = WA.at[0:48, 0:_DEG * _H].set(WA48)
    WA = WA.at[48:54, _DEG * _H:].set(Wl6[0:6])
    def build_wd(nrow0, pln, cil, cjl):
        M = np.zeros((57, 54), np.float32)
        for n in range(_DEG):
            M[nrow0 + n, n] = -1.0
            M[pln, n] += 1.0
            M[nrow0 + n, 16 + n] = -1.0
            M[cil, 16 + n] += 1.0
            M[nrow0 + n, 32 + n] = -1.0
            M[cjl, 32 + n] += 1.0
        M[pln, 48] += 1.0
        M[pln, 49] += 1.0
        M[cil, 49] -= 1.0
        M[pln, 50] += 1.0
        M[cjl, 50] -= 1.0
        M[cil, 51] += 1.0
        M[cil, 52] += 1.0
        M[cjl, 52] -= 1.0
        M[cjl, 53] += 1.0
        return jnp.asarray(M)

    pad10 = ((0, 0), (0, 10))
    WD = jnp.concatenate([
        jnp.pad(build_wd(0, 48, 51, 54), pad10),
        jnp.pad(build_wd(16, 49, 52, 55), pad10),
        jnp.pad(build_wd(32, 50, 53, 56), pad10)], axis=1)
    in_specs, out_specs, out_shape = _tc_specs()
    h_out, coord_out = pl.pallas_call(
        _tc_body,
        grid=(_GRID,),
        in_specs=in_specs,
        out_specs=out_specs,
        out_shape=out_shape,
        compiler_params=pltpu.CompilerParams(
            dimension_semantics=("parallel",)),
        scratch_shapes=[
            pltpu.VMEM((_BE, _DEG * _H), jnp.float32),
            pltpu.VMEM((_BE, _H), jnp.float32),
        ],
    )(h, coord, cc.T,
      ncx, ncy, ncz, hc, edge_attr,
      We1a, We1b, We1g, We1e, be1[None], We2, be2[None],
      Wc1, bc1[None], Wc2, WA, WD, w[0], Wn1[:_F], Wn1[_F:],
      bn1[None], Wn2, bn2[None])
    return h_out, coord_out
